# Initial kernel scaffold; baseline (speedup 1.0000x reference)
import jax
import jax.numpy as jnp
from jax.experimental import pallas as pl


def kernel(x, edge_index, edge_attr, batch, W_ne, b_ne, W_ee, b_ee, lin_w0, att_src0, att_dst0, att_edge0, lin_edge_w0, conv_b0, bn_g0, bn_b0, lin_w1, att_src1, att_dst1, att_edge1, lin_edge_w1, conv_b1, bn_g1, bn_b1, W_c1, b_c1, W_c2, b_c2):
    raise NotImplementedError("write your pallas kernel here")


# speedup vs baseline: 1.0594x; 1.0594x over previous
import jax
import jax.numpy as jnp
from jax.experimental import pallas as pl

N = 50000
E = 800000
IN = 128
HID = 64
H = 4
C = 16
G = 64


def _gat_edge_phase(xl, asrc, adst, ae, src, dst):
    alpha = asrc[src] + adst[dst] + ae
    alpha = jnp.where(alpha > 0, alpha, 0.2 * alpha)
    ex = jnp.exp(alpha)
    den = jax.ops.segment_sum(ex, dst, num_segments=N)
    msg = xl.reshape(N, H, C)[src] * ex[:, :, None]
    num = jax.ops.segment_sum(msg, dst, num_segments=N)
    out = num / (den[:, :, None] + 1e-16)
    return out.reshape(N, H * C)


def kernel(x, edge_index, edge_attr, batch, W_ne, b_ne, W_ee, b_ee, lin_w0, att_src0, att_dst0, att_edge0, lin_edge_w0, conv_b0, bn_g0, bn_b0, lin_w1, att_src1, att_dst1, att_edge1, lin_edge_w1, conv_b1, bn_g1, bn_b1, W_c1, b_c1, W_c2, b_c2):
    src = edge_index[0]
    dst = edge_index[1]

    def prep(lin_w, a_s, a_d, a_e, lew):
        W3 = lin_w.reshape(H, C, HID)
        As = jnp.einsum("hck,hc->kh", W3, a_s)
        Ad = jnp.einsum("hck,hc->kh", W3, a_d)
        B3 = lew.reshape(H, C, HID)
        Bm = jnp.einsum("hck,hc->kh", B3, a_e)
        u = W_ee[:, 0] @ Bm
        v = b_ee @ Bm
        return As, Ad, u, v

    bs = 1.0 / jnp.sqrt(1.0 + 1e-5)

    h = x @ W_ne.T + b_ne
    layer_params = [
        (lin_w0, att_src0, att_dst0, att_edge0, lin_edge_w0, conv_b0, bn_g0, bn_b0),
        (lin_w1, att_src1, att_dst1, att_edge1, lin_edge_w1, conv_b1, bn_g1, bn_b1),
    ]
    for (lw, a_s, a_d, a_e, lew, cb, g, b) in layer_params:
        As, Ad, u, v = prep(lw, a_s, a_d, a_e, lew)
        xl = h @ lw.T
        asrc = h @ As
        adst = h @ Ad
        ae = edge_attr * u[None, :] + v[None, :]
        gat = _gat_edge_phase(xl, asrc, adst, ae, src, dst)
        scale = bs * g
        bias = cb * scale + b
        h2 = gat * scale + bias
        h = jnp.where(h2 > 0, h2, 0.0) + h

    sums = jax.ops.segment_sum(h, batch, num_segments=G)
    cnt = jax.ops.segment_sum(jnp.ones((N, 1), jnp.float32), batch, num_segments=G)
    pooled = sums / jnp.maximum(cnt, 1.0)
    hc = jax.nn.relu(pooled @ W_c1.T + b_c1)
    out = jax.nn.sigmoid(hc @ W_c2.T + b_c2)
    return out.squeeze(-1)


# speedup vs baseline: 14.3682x; 13.5625x over previous
import functools

import jax
import jax.numpy as jnp
from jax import lax
from jax.experimental import pallas as pl
from jax.experimental.pallas import tpu as pltpu
from jax.experimental.pallas import tpu_sc as plsc

N = 50000
E = 800000
IN = 128
HID = 64
H = 4
C = 16
G = 64

NSUB = 16
NCORE = 2
N2 = N // 2
NQ = N // 4
R2 = 12544
SENTQ = NQ
EPW = E // NSUB
K = 400
KP = 512
NCH = EPW // K

_NB = 125
_BN = 400


def _enc_dense_body(x_ref, wne_ref, bne_ref, lwT_ref, as_ref, ad_ref,
                    h_ref, xl_ref, asrc_ref, adst_ref):
    h = jnp.dot(x_ref[...], wne_ref[...], preferred_element_type=jnp.float32)
    h = h + bne_ref[...]
    h_ref[...] = h
    xl_ref[...] = jnp.dot(h, lwT_ref[...], preferred_element_type=jnp.float32)
    asrc_ref[...] = jnp.dot(h, as_ref[...], preferred_element_type=jnp.float32)
    adst_ref[...] = jnp.dot(h, ad_ref[...], preferred_element_type=jnp.float32)


def _epi_dense_body(num_ref, den_ref, hin_ref, sc_ref, bi_ref,
                    lwT_ref, as_ref, ad_ref,
                    h_ref, xl_ref, asrc_ref, adst_ref):
    heads = lax.broadcasted_iota(jnp.int32, (H, HID), 1) // C
    R = (heads == lax.broadcasted_iota(jnp.int32, (H, HID), 0)).astype(jnp.float32)
    denx = jnp.dot(den_ref[...], R, preferred_element_type=jnp.float32)
    gat = num_ref[...] / (denx + 1e-16)
    h2 = gat * sc_ref[...] + bi_ref[...]
    h = jnp.maximum(h2, 0.0) + hin_ref[...]
    h_ref[...] = h
    xl_ref[...] = jnp.dot(h, lwT_ref[...], preferred_element_type=jnp.float32)
    asrc_ref[...] = jnp.dot(h, as_ref[...], preferred_element_type=jnp.float32)
    adst_ref[...] = jnp.dot(h, ad_ref[...], preferred_element_type=jnp.float32)


def _final_body(num_ref, den_ref, hin_ref, sc_ref, bi_ref, batch_ref,
                wc1_ref, bc1_ref, wc2_ref, bc2_ref,
                out_ref, sums_ref, cnt_ref):
    i = pl.program_id(0)
    heads = lax.broadcasted_iota(jnp.int32, (H, HID), 1) // C
    R = (heads == lax.broadcasted_iota(jnp.int32, (H, HID), 0)).astype(jnp.float32)
    denx = jnp.dot(den_ref[...], R, preferred_element_type=jnp.float32)
    gat = num_ref[...] / (denx + 1e-16)
    h2 = gat * sc_ref[...] + bi_ref[...]
    h = jnp.maximum(h2, 0.0) + hin_ref[...]

    b = batch_ref[0, 0, :]
    oh = (b[:, None] == lax.broadcasted_iota(jnp.int32, (_BN, G), 1)).astype(jnp.float32)
    sums_d = lax.dot_general(oh, h, (((0,), (0,)), ((), ())),
                             preferred_element_type=jnp.float32)
    cnt_d = lax.dot_general(oh, jnp.ones((_BN, HID), jnp.float32),
                            (((0,), (0,)), ((), ())),
                            preferred_element_type=jnp.float32)

    @pl.when(i == 0)
    def _():
        sums_ref[...] = sums_d
        cnt_ref[...] = cnt_d

    @pl.when(i > 0)
    def _():
        sums_ref[...] += sums_d
        cnt_ref[...] += cnt_d

    @pl.when(i == _NB - 1)
    def _():
        pooled = sums_ref[...] / jnp.maximum(cnt_ref[...], 1.0)
        hc = jnp.dot(pooled, wc1_ref[...], preferred_element_type=jnp.float32)
        hc = jnp.maximum(hc + bc1_ref[...], 0.0)
        o = jnp.dot(hc, wc2_ref[...], preferred_element_type=jnp.float32)
        o = jax.nn.sigmoid(o + bc2_ref[...])
        out_ref[...] = o.T


def _edge_body(src_h, dst_h, ea_h, asrc_h, adst_h, xl_h, ut_h, vt_h,
               zn_h, zd_h, ninf_h,
               num_h, den_h,
               S_asrc, S_adst, S_num, S_den,
               src_v, dst_v, ea_v, idxa_v, idxd_v, idxe_v,
               ag_v, bg_v, ex_v, xlG, uv_v, vv_v,
               sem0, sem1, sem2):
    cid = lax.axis_index("c")
    sid = lax.axis_index("s")

    @pl.when(sid < NSUB - 1)
    def _():
        pltpu.sync_copy(asrc_h.at[pl.ds(sid * 12512, 12512)],
                        S_asrc.at[pl.ds(sid * 12512, 12512)])

    @pl.when(sid == NSUB - 1)
    def _():
        pltpu.sync_copy(asrc_h.at[pl.ds(15 * 12512, 12320)],
                        S_asrc.at[pl.ds(15 * 12512, 12320)])

    pltpu.sync_copy(ut_h, uv_v)
    pltpu.sync_copy(vt_h, vv_v)

    for t in range(K, KP, 16):
        idxd_v[pl.ds(t, 16)] = jnp.full((16,), SENTQ, jnp.int32)
    for t in range(K * H, KP * H, 16):
        idxe_v[pl.ds(t, 16)] = jnp.full((16,), SENTQ * H, jnp.int32)

    utile = uv_v[...]
    vtile = vv_v[...]
    expand4 = lax.iota(jnp.int32, 16) // jnp.int32(4)
    headpat = lax.iota(jnp.int32, 16) % jnp.int32(4)

    for p in range(2):
        qb = cid * N2 + p * NQ

        @pl.when(sid < NSUB - 1)
        def _():
            pltpu.sync_copy(adst_h.at[pl.ds(qb * H + sid * 3136, 3136)],
                            S_adst.at[pl.ds(sid * 3136, 3136)])

        @pl.when(sid == NSUB - 1)
        def _():
            pltpu.sync_copy(adst_h.at[pl.ds(qb * H + 15 * 3136, 2960)],
                            S_adst.at[pl.ds(15 * 3136, 2960)])

        @pl.when(sid == 0)
        def _():
            pltpu.sync_copy(ninf_h, S_adst.at[pl.ds(NQ * H, (R2 - NQ) * H)])

        pltpu.sync_copy(zn_h, S_num.at[pl.ds(sid * 784, 784)])
        pltpu.sync_copy(zd_h, S_den.at[pl.ds(sid * 3136, 3136)])

        plsc.subcore_barrier()

        def chunk(j, carry):
            base = pl.multiple_of(sid * EPW + j * K, 8)
            pltpu.sync_copy(src_h.at[pl.ds(base, K)], src_v)
            pltpu.sync_copy(dst_h.at[pl.ds(base, K)], dst_v)
            pltpu.sync_copy(ea_h.at[pl.ds(base, K)], ea_v)

            qbv = jnp.full((16,), qb, jnp.int32)

            def route(io, c2):
                sv = src_v[pl.ds(io * 16, 16)]
                d = dst_v[pl.ds(io * 16, 16)]
                own = (d >= qbv) & (d < qbv + jnp.int32(NQ))
                loc = jnp.where(own, d - qbv, jnp.full((16,), SENTQ, jnp.int32))
                idxd_v[pl.ds(io * 16, 16)] = loc
                for q in range(4):
                    idxc = expand4 + jnp.int32(4 * q)
                    sexp = jnp.take_along_axis(sv, idxc, axis=0,
                                               mode="promise_in_bounds")
                    lexp = jnp.take_along_axis(loc, idxc, axis=0,
                                               mode="promise_in_bounds")
                    j0 = io * 64 + q * 16
                    idxa_v[pl.ds(j0, 16)] = sexp * H + headpat
                    idxe_v[pl.ds(j0, 16)] = lexp * H + headpat
                return c2
            lax.fori_loop(0, K // 16, route, 0)

            cp0 = pltpu.async_copy(S_asrc.at[idxa_v], ag_v, sem0)
            cp1 = pltpu.async_copy(S_adst.at[idxe_v], bg_v, sem1)
            cp2 = pltpu.async_copy(xl_h.at[src_v], xlG.at[pl.ds(0, K)], sem2)
            cp0.wait()
            cp1.wait()
            cp2.wait()

            def exloop(io, c2):
                eav = ea_v[pl.ds(io * 16, 16)]
                for q in range(4):
                    idxc = expand4 + jnp.int32(4 * q)
                    aeq = jnp.take_along_axis(eav, idxc, axis=0,
                                              mode="promise_in_bounds")
                    aeq = aeq * utile + vtile
                    j0 = io * 64 + q * 16
                    a = ag_v[pl.ds(j0, 16)] + bg_v[pl.ds(j0, 16)] + aeq
                    a = jnp.maximum(a, a * 0.2)
                    ex_v[pl.ds(j0, 16)] = jnp.exp(a)
                return c2
            lax.fori_loop(0, K // 16, exloop, 0)

            def mulloop(io, c2):
                for q in range(4):
                    exq = ex_v[pl.ds(io * 64 + q * 16, 16)]
                    for t in range(4):
                        e = io * 16 + q * 4 + t
                        for hh in range(H):
                            splat = jnp.take_along_axis(
                                exq, jnp.full((16,), t * 4 + hh, jnp.int32),
                                axis=0, mode="promise_in_bounds")
                            v = xlG[e, pl.ds(hh * 16, 16)]
                            xlG[e, pl.ds(hh * 16, 16)] = v * splat
                return c2
            lax.fori_loop(0, K // 16, mulloop, 0)

            pltpu.sync_copy(ex_v, S_den.at[idxe_v], add=True)
            pltpu.sync_copy(xlG, S_num.at[idxd_v], add=True)
            return carry

        lax.fori_loop(0, NCH, chunk, 0)

        plsc.subcore_barrier()

        @pl.when(sid < NSUB - 1)
        def _():
            pltpu.sync_copy(S_num.at[pl.ds(sid * 784, 784)],
                            num_h.at[pl.ds(qb + sid * 784, 784)])
            pltpu.sync_copy(S_den.at[pl.ds(sid * 3136, 3136)],
                            den_h.at[pl.ds(qb * H + sid * 3136, 3136)])

        @pl.when(sid == NSUB - 1)
        def _():
            pltpu.sync_copy(S_num.at[pl.ds(15 * 784, 740)],
                            num_h.at[pl.ds(qb + 15 * 784, 740)])
            pltpu.sync_copy(S_den.at[pl.ds(15 * 3136, 2960)],
                            den_h.at[pl.ds(qb * H + 15 * 3136, 2960)])

        plsc.subcore_barrier()


_edge_kernel = functools.partial(
    pl.kernel,
    _edge_body,
    out_type=(jax.ShapeDtypeStruct((N, HID), jnp.float32),
              jax.ShapeDtypeStruct((N * H,), jnp.float32)),
    mesh=plsc.VectorSubcoreMesh(core_axis_name="c", subcore_axis_name="s"),
    compiler_params=pltpu.CompilerParams(needs_layout_passes=False,
                                         use_tc_tiling_on_sc=False),
    scratch_types=(
        pltpu.VMEM_SHARED((N * H,), jnp.float32),
        pltpu.VMEM_SHARED((R2 * H,), jnp.float32),
        pltpu.VMEM_SHARED((R2, HID), jnp.float32),
        pltpu.VMEM_SHARED((R2 * H,), jnp.float32),
        pltpu.VMEM((K,), jnp.int32),
        pltpu.VMEM((K,), jnp.int32),
        pltpu.VMEM((K,), jnp.float32),
        pltpu.VMEM((K * H,), jnp.int32),
        pltpu.VMEM((KP,), jnp.int32),
        pltpu.VMEM((KP * H,), jnp.int32),
        pltpu.VMEM((K * H,), jnp.float32),
        pltpu.VMEM((KP * H,), jnp.float32),
        pltpu.VMEM((KP * H,), jnp.float32),
        pltpu.VMEM((KP, HID), jnp.float32),
        pltpu.VMEM((16,), jnp.float32),
        pltpu.VMEM((16,), jnp.float32),
        pltpu.SemaphoreType.DMA,
        pltpu.SemaphoreType.DMA,
        pltpu.SemaphoreType.DMA,
    ),
)


def kernel(x, edge_index, edge_attr, batch, W_ne, b_ne, W_ee, b_ee, lin_w0, att_src0, att_dst0, att_edge0, lin_edge_w0, conv_b0, bn_g0, bn_b0, lin_w1, att_src1, att_dst1, att_edge1, lin_edge_w1, conv_b1, bn_g1, bn_b1, W_c1, b_c1, W_c2, b_c2):
    f32 = jnp.float32
    src = edge_index[0]
    dst = edge_index[1]
    eaf = edge_attr.reshape(E)

    def prep(lin_w, a_s, a_d, a_e, lew):
        W3 = lin_w.reshape(H, C, HID)
        As = jnp.einsum("hck,hc->kh", W3, a_s)
        Ad = jnp.einsum("hck,hc->kh", W3, a_d)
        B3 = lew.reshape(H, C, HID)
        Bm = jnp.einsum("hck,hc->kh", B3, a_e)
        u = W_ee[:, 0] @ Bm
        v = b_ee @ Bm
        return As, Ad, jnp.tile(u, H), jnp.tile(v, H)

    bs = 1.0 / jnp.sqrt(1.0 + 1e-5)
    As0, Ad0, ut0, vt0 = prep(lin_w0, att_src0, att_dst0, att_edge0, lin_edge_w0)
    As1, Ad1, ut1, vt1 = prep(lin_w1, att_src1, att_dst1, att_edge1, lin_edge_w1)
    sc0 = (bs * bn_g0).reshape(1, HID)
    bi0 = (conv_b0 * bs * bn_g0 + bn_b0).reshape(1, HID)
    sc1 = (bs * bn_g1).reshape(1, HID)
    bi1 = (conv_b1 * bs * bn_g1 + bn_b1).reshape(1, HID)

    zn = jnp.zeros((784, HID), f32)
    zd = jnp.zeros((784 * H,), f32)
    ninf = jnp.full(((R2 - NQ) * H,), -jnp.inf, f32)

    h0, xl0, asrc0, adst0 = pl.pallas_call(
        _enc_dense_body,
        grid=(_NB,),
        in_specs=[
            pl.BlockSpec((_BN, IN), lambda i: (i, 0)),
            pl.BlockSpec((IN, HID), lambda i: (0, 0)),
            pl.BlockSpec((1, HID), lambda i: (0, 0)),
            pl.BlockSpec((HID, HID), lambda i: (0, 0)),
            pl.BlockSpec((HID, H), lambda i: (0, 0)),
            pl.BlockSpec((HID, H), lambda i: (0, 0)),
        ],
        out_specs=[
            pl.BlockSpec((_BN, HID), lambda i: (i, 0)),
            pl.BlockSpec((_BN, HID), lambda i: (i, 0)),
            pl.BlockSpec((_BN, H), lambda i: (i, 0)),
            pl.BlockSpec((_BN, H), lambda i: (i, 0)),
        ],
        out_shape=[
            jax.ShapeDtypeStruct((N, HID), f32),
            jax.ShapeDtypeStruct((N, HID), f32),
            jax.ShapeDtypeStruct((N, H), f32),
            jax.ShapeDtypeStruct((N, H), f32),
        ],
    )(x, W_ne.T, b_ne.reshape(1, HID), lin_w0.T, As0, Ad0)

    num0, den0 = _edge_kernel()(src, dst, eaf,
                                asrc0.reshape(N * H), adst0.reshape(N * H),
                                xl0, ut0, vt0, zn, zd, ninf)
    den0 = den0.reshape(N, H)

    h1, xl1, asrc1, adst1 = pl.pallas_call(
        _epi_dense_body,
        grid=(_NB,),
        in_specs=[
            pl.BlockSpec((_BN, HID), lambda i: (i, 0)),
            pl.BlockSpec((_BN, H), lambda i: (i, 0)),
            pl.BlockSpec((_BN, HID), lambda i: (i, 0)),
            pl.BlockSpec((1, HID), lambda i: (0, 0)),
            pl.BlockSpec((1, HID), lambda i: (0, 0)),
            pl.BlockSpec((HID, HID), lambda i: (0, 0)),
            pl.BlockSpec((HID, H), lambda i: (0, 0)),
            pl.BlockSpec((HID, H), lambda i: (0, 0)),
        ],
        out_specs=[
            pl.BlockSpec((_BN, HID), lambda i: (i, 0)),
            pl.BlockSpec((_BN, HID), lambda i: (i, 0)),
            pl.BlockSpec((_BN, H), lambda i: (i, 0)),
            pl.BlockSpec((_BN, H), lambda i: (i, 0)),
        ],
        out_shape=[
            jax.ShapeDtypeStruct((N, HID), f32),
            jax.ShapeDtypeStruct((N, HID), f32),
            jax.ShapeDtypeStruct((N, H), f32),
            jax.ShapeDtypeStruct((N, H), f32),
        ],
    )(num0, den0, h0, sc0, bi0, lin_w1.T, As1, Ad1)

    num1, den1 = _edge_kernel()(src, dst, eaf,
                                asrc1.reshape(N * H), adst1.reshape(N * H),
                                xl1, ut1, vt1, zn, zd, ninf)
    den1 = den1.reshape(N, H)

    out, _sums, _cnt = pl.pallas_call(
        _final_body,
        grid=(_NB,),
        in_specs=[
            pl.BlockSpec((_BN, HID), lambda i: (i, 0)),
            pl.BlockSpec((_BN, H), lambda i: (i, 0)),
            pl.BlockSpec((_BN, HID), lambda i: (i, 0)),
            pl.BlockSpec((1, HID), lambda i: (0, 0)),
            pl.BlockSpec((1, HID), lambda i: (0, 0)),
            pl.BlockSpec((1, 1, _BN), lambda i: (i, 0, 0)),
            pl.BlockSpec((HID, HID // 2), lambda i: (0, 0)),
            pl.BlockSpec((1, HID // 2), lambda i: (0, 0)),
            pl.BlockSpec((HID // 2, 1), lambda i: (0, 0)),
            pl.BlockSpec((1, 1), lambda i: (0, 0)),
        ],
        out_specs=[
            pl.BlockSpec((1, G), lambda i: (0, 0)),
            pl.BlockSpec((G, HID), lambda i: (0, 0)),
            pl.BlockSpec((G, HID), lambda i: (0, 0)),
        ],
        out_shape=[
            jax.ShapeDtypeStruct((1, G), f32),
            jax.ShapeDtypeStruct((G, HID), f32),
            jax.ShapeDtypeStruct((G, HID), f32),
        ],
    )(num1, den1, h1, sc1, bi1, batch.reshape(_NB, 1, _BN),
      W_c1.T, b_c1.reshape(1, HID // 2), W_c2.T, b_c2.reshape(1, 1))

    return out.reshape(G)


# speedup vs baseline: 55.2840x; 3.8477x over previous
import functools

import jax
import jax.numpy as jnp
from jax import lax
from jax.experimental import pallas as pl
from jax.experimental.pallas import tpu as pltpu
from jax.experimental.pallas import tpu_sc as plsc

N = 50000
E = 800000
IN = 128
HID = 64
H = 4
C = 16
G = 64

NSUB = 16
NCORE = 2
N2 = N // 2
NQ = N // 4
R2 = 12544
SENTQ = NQ
EPW = E // NSUB
K = 400
KP = 512
NCH = EPW // K

_NB = 125
_BN = 400


def _enc_dense_body(x_ref, wne_ref, bne_ref, lwT_ref, as_ref, ad_ref,
                    h_ref, xl_ref, asrc_ref, adst_ref):
    h = jnp.dot(x_ref[...], wne_ref[...], preferred_element_type=jnp.float32)
    h = h + bne_ref[...]
    h_ref[...] = h
    xl_ref[...] = jnp.dot(h, lwT_ref[...], preferred_element_type=jnp.float32)
    asrc_ref[...] = jnp.dot(h, as_ref[...], preferred_element_type=jnp.float32)
    adst_ref[...] = jnp.dot(h, ad_ref[...], preferred_element_type=jnp.float32)


def _epi_dense_body(num_ref, den_ref, hin_ref, sc_ref, bi_ref,
                    lwT_ref, as_ref, ad_ref,
                    h_ref, xl_ref, asrc_ref, adst_ref):
    heads = lax.broadcasted_iota(jnp.int32, (H, HID), 1) // C
    R = (heads == lax.broadcasted_iota(jnp.int32, (H, HID), 0)).astype(jnp.float32)
    denx = jnp.dot(den_ref[...], R, preferred_element_type=jnp.float32)
    gat = num_ref[...] / (denx + 1e-16)
    h2 = gat * sc_ref[...] + bi_ref[...]
    h = jnp.maximum(h2, 0.0) + hin_ref[...]
    h_ref[...] = h
    xl_ref[...] = jnp.dot(h, lwT_ref[...], preferred_element_type=jnp.float32)
    asrc_ref[...] = jnp.dot(h, as_ref[...], preferred_element_type=jnp.float32)
    adst_ref[...] = jnp.dot(h, ad_ref[...], preferred_element_type=jnp.float32)


def _final_body(num_ref, den_ref, hin_ref, sc_ref, bi_ref, batch_ref,
                wc1_ref, bc1_ref, wc2_ref, bc2_ref,
                out_ref, sums_ref, cnt_ref):
    i = pl.program_id(0)
    heads = lax.broadcasted_iota(jnp.int32, (H, HID), 1) // C
    R = (heads == lax.broadcasted_iota(jnp.int32, (H, HID), 0)).astype(jnp.float32)
    denx = jnp.dot(den_ref[...], R, preferred_element_type=jnp.float32)
    gat = num_ref[...] / (denx + 1e-16)
    h2 = gat * sc_ref[...] + bi_ref[...]
    h = jnp.maximum(h2, 0.0) + hin_ref[...]

    b = batch_ref[0, 0, :]
    oh = (b[:, None] == lax.broadcasted_iota(jnp.int32, (_BN, G), 1)).astype(jnp.float32)
    sums_d = lax.dot_general(oh, h, (((0,), (0,)), ((), ())),
                             preferred_element_type=jnp.float32)
    cnt_d = lax.dot_general(oh, jnp.ones((_BN, HID), jnp.float32),
                            (((0,), (0,)), ((), ())),
                            preferred_element_type=jnp.float32)

    @pl.when(i == 0)
    def _():
        sums_ref[...] = sums_d
        cnt_ref[...] = cnt_d

    @pl.when(i > 0)
    def _():
        sums_ref[...] += sums_d
        cnt_ref[...] += cnt_d

    @pl.when(i == _NB - 1)
    def _():
        pooled = sums_ref[...] / jnp.maximum(cnt_ref[...], 1.0)
        hc = jnp.dot(pooled, wc1_ref[...], preferred_element_type=jnp.float32)
        hc = jnp.maximum(hc + bc1_ref[...], 0.0)
        o = jnp.dot(hc, wc2_ref[...], preferred_element_type=jnp.float32)
        o = jax.nn.sigmoid(o + bc2_ref[...])
        out_ref[...] = o.T


KR = K + 16


def _edge_body(src_h, dst_h, ea_h, asrc_h, adst_h, xl_h, ut_h, vt_h,
               zn_h, zd_h, ninf_h,
               num_h, den_h,
               S_asrc, S_adst, S_num, S_den,
               src_v, dst_v, ea_v, srcC, locC, eaC,
               idxs_v, idxa_v, idxd_v, idxe_v,
               ag_v, bg_v, ex_v, xlG, uv_v, vv_v,
               sem0, sem1, sem2):
    cid = lax.axis_index("c")
    sid = lax.axis_index("s")

    @pl.when(sid < NSUB - 1)
    def _():
        pltpu.sync_copy(asrc_h.at[pl.ds(sid * 12512, 12512)],
                        S_asrc.at[pl.ds(sid * 12512, 12512)])

    @pl.when(sid == NSUB - 1)
    def _():
        pltpu.sync_copy(asrc_h.at[pl.ds(15 * 12512, 12320)],
                        S_asrc.at[pl.ds(15 * 12512, 12320)])

    pltpu.sync_copy(ut_h, uv_v)
    pltpu.sync_copy(vt_h, vv_v)

    utile = uv_v[...]
    vtile = vv_v[...]
    expand4 = lax.iota(jnp.int32, 16) // jnp.int32(4)
    headpat = lax.iota(jnp.int32, 16) % jnp.int32(4)
    lanes16 = lax.iota(jnp.int32, 16)

    def drain():
        def build(io, c2):
            sv = srcC[pl.ds(io * 16, 16)]
            lv = locC[pl.ds(io * 16, 16)]
            idxs_v[pl.ds(io * 16, 16)] = sv
            idxd_v[pl.ds(io * 16, 16)] = lv
            for q in range(4):
                idxc = expand4 + jnp.int32(4 * q)
                sexp = jnp.take_along_axis(sv, idxc, axis=0,
                                           mode="promise_in_bounds")
                lexp = jnp.take_along_axis(lv, idxc, axis=0,
                                           mode="promise_in_bounds")
                j0 = io * 64 + q * 16
                idxa_v[pl.ds(j0, 16)] = sexp * H + headpat
                idxe_v[pl.ds(j0, 16)] = lexp * H + headpat
            return c2
        lax.fori_loop(0, K // 16, build, 0)

        cp0 = pltpu.async_copy(S_asrc.at[idxa_v], ag_v, sem0)
        cp1 = pltpu.async_copy(S_adst.at[idxe_v], bg_v, sem1)
        cp2 = pltpu.async_copy(xl_h.at[idxs_v], xlG, sem2)
        cp0.wait()
        cp1.wait()
        cp2.wait()

        def exloop(io, c2):
            eav = eaC[pl.ds(io * 16, 16)]
            for q in range(4):
                idxc = expand4 + jnp.int32(4 * q)
                aeq = jnp.take_along_axis(eav, idxc, axis=0,
                                          mode="promise_in_bounds")
                aeq = aeq * utile + vtile
                j0 = io * 64 + q * 16
                a = ag_v[pl.ds(j0, 16)] + bg_v[pl.ds(j0, 16)] + aeq
                a = jnp.maximum(a, a * 0.2)
                ex_v[pl.ds(j0, 16)] = jnp.exp(a)
            return c2
        lax.fori_loop(0, K // 16, exloop, 0)

        def mulloop(io, c2):
            for q in range(4):
                exq = ex_v[pl.ds(io * 64 + q * 16, 16)]
                for t in range(4):
                    e = io * 16 + q * 4 + t
                    for hh in range(H):
                        splat = jnp.take_along_axis(
                            exq, jnp.full((16,), t * 4 + hh, jnp.int32),
                            axis=0, mode="promise_in_bounds")
                        v = xlG[e, pl.ds(hh * 16, 16)]
                        xlG[e, pl.ds(hh * 16, 16)] = v * splat
            return c2
        lax.fori_loop(0, K // 16, mulloop, 0)

        pltpu.sync_copy(ex_v, S_den.at[idxe_v], add=True)
        pltpu.sync_copy(xlG, S_num.at[idxd_v], add=True)

    for p in range(2):
        qb = cid * N2 + p * NQ

        @pl.when(sid < NSUB - 1)
        def _():
            pltpu.sync_copy(adst_h.at[pl.ds(qb * H + sid * 3136, 3136)],
                            S_adst.at[pl.ds(sid * 3136, 3136)])

        @pl.when(sid == NSUB - 1)
        def _():
            pltpu.sync_copy(adst_h.at[pl.ds(qb * H + 15 * 3136, 2960)],
                            S_adst.at[pl.ds(15 * 3136, 2960)])

        @pl.when(sid == 0)
        def _():
            pltpu.sync_copy(ninf_h, S_adst.at[pl.ds(NQ * H, (R2 - NQ) * H)])

        pltpu.sync_copy(zn_h, S_num.at[pl.ds(sid * 784, 784)])
        pltpu.sync_copy(zd_h, S_den.at[pl.ds(sid * 3136, 3136)])

        plsc.subcore_barrier()

        qbv = jnp.full((16,), qb, jnp.int32)

        def chunk(j, fill):
            base = pl.multiple_of(sid * EPW + j * K, 8)
            pltpu.sync_copy(src_h.at[pl.ds(base, K)], src_v)
            pltpu.sync_copy(dst_h.at[pl.ds(base, K)], dst_v)
            pltpu.sync_copy(ea_h.at[pl.ds(base, K)], ea_v)

            def route(io, f):
                sv = src_v[pl.ds(io * 16, 16)]
                d = dst_v[pl.ds(io * 16, 16)]
                eav = ea_v[pl.ds(io * 16, 16)]
                own = (d >= qbv) & (d < qbv + jnp.int32(NQ))
                loc = d - qbv
                plsc.store_compressed(srcC.at[pl.ds(f, 16)], sv, mask=own)
                plsc.store_compressed(locC.at[pl.ds(f, 16)], loc, mask=own)
                plsc.store_compressed(eaC.at[pl.ds(f, 16)], eav, mask=own)
                cnt = plsc.all_reduce_population_count(own)
                f = f + (cnt[0] if getattr(cnt, "ndim", 0) else cnt)

                def do_drain(ff):
                    drain()
                    srcC[pl.ds(0, 16)] = srcC[pl.ds(K, 16)]
                    locC[pl.ds(0, 16)] = locC[pl.ds(K, 16)]
                    eaC[pl.ds(0, 16)] = eaC[pl.ds(K, 16)]
                    return ff - jnp.int32(K)

                return lax.cond(f >= K, do_drain, lambda ff: ff, f)
            return lax.fori_loop(0, K // 16, route, fill)

        fill = lax.fori_loop(0, NCH, chunk, jnp.int32(0))

        fillv = jnp.full((16,), 0, jnp.int32) + fill

        def pad(io, c2):
            pos = lanes16 + jnp.int32(io * 16)
            m = pos >= fillv
            sv = srcC[pl.ds(io * 16, 16)]
            lv = locC[pl.ds(io * 16, 16)]
            ev = eaC[pl.ds(io * 16, 16)]
            srcC[pl.ds(io * 16, 16)] = jnp.where(m, jnp.zeros((16,), jnp.int32), sv)
            locC[pl.ds(io * 16, 16)] = jnp.where(
                m, jnp.full((16,), SENTQ, jnp.int32), lv)
            eaC[pl.ds(io * 16, 16)] = jnp.where(m, jnp.zeros((16,), jnp.float32), ev)
            return c2
        lax.fori_loop(0, K // 16, pad, 0)
        drain()

        plsc.subcore_barrier()

        @pl.when(sid < NSUB - 1)
        def _():
            pltpu.sync_copy(S_num.at[pl.ds(sid * 784, 784)],
                            num_h.at[pl.ds(qb + sid * 784, 784)])
            pltpu.sync_copy(S_den.at[pl.ds(sid * 3136, 3136)],
                            den_h.at[pl.ds(qb * H + sid * 3136, 3136)])

        @pl.when(sid == NSUB - 1)
        def _():
            pltpu.sync_copy(S_num.at[pl.ds(15 * 784, 740)],
                            num_h.at[pl.ds(qb + 15 * 784, 740)])
            pltpu.sync_copy(S_den.at[pl.ds(15 * 3136, 2960)],
                            den_h.at[pl.ds(qb * H + 15 * 3136, 2960)])

        plsc.subcore_barrier()


_edge_kernel = functools.partial(
    pl.kernel,
    _edge_body,
    out_type=(jax.ShapeDtypeStruct((N, HID), jnp.float32),
              jax.ShapeDtypeStruct((N * H,), jnp.float32)),
    mesh=plsc.VectorSubcoreMesh(core_axis_name="c", subcore_axis_name="s"),
    compiler_params=pltpu.CompilerParams(needs_layout_passes=False,
                                         use_tc_tiling_on_sc=False),
    scratch_types=(
        pltpu.VMEM_SHARED((N * H,), jnp.float32),
        pltpu.VMEM_SHARED((R2 * H,), jnp.float32),
        pltpu.VMEM_SHARED((R2, HID), jnp.float32),
        pltpu.VMEM_SHARED((R2 * H,), jnp.float32),
        pltpu.VMEM((K,), jnp.int32),
        pltpu.VMEM((K,), jnp.int32),
        pltpu.VMEM((K,), jnp.float32),
        pltpu.VMEM((KR,), jnp.int32),
        pltpu.VMEM((KR,), jnp.int32),
        pltpu.VMEM((KR,), jnp.float32),
        pltpu.VMEM((K,), jnp.int32),
        pltpu.VMEM((K * H,), jnp.int32),
        pltpu.VMEM((K,), jnp.int32),
        pltpu.VMEM((K * H,), jnp.int32),
        pltpu.VMEM((K * H,), jnp.float32),
        pltpu.VMEM((K * H,), jnp.float32),
        pltpu.VMEM((K * H,), jnp.float32),
        pltpu.VMEM((K, HID), jnp.float32),
        pltpu.VMEM((16,), jnp.float32),
        pltpu.VMEM((16,), jnp.float32),
        pltpu.SemaphoreType.DMA,
        pltpu.SemaphoreType.DMA,
        pltpu.SemaphoreType.DMA,
    ),
)


def kernel(x, edge_index, edge_attr, batch, W_ne, b_ne, W_ee, b_ee, lin_w0, att_src0, att_dst0, att_edge0, lin_edge_w0, conv_b0, bn_g0, bn_b0, lin_w1, att_src1, att_dst1, att_edge1, lin_edge_w1, conv_b1, bn_g1, bn_b1, W_c1, b_c1, W_c2, b_c2):
    f32 = jnp.float32
    src = edge_index[0]
    dst = edge_index[1]
    eaf = edge_attr.reshape(E)

    def prep(lin_w, a_s, a_d, a_e, lew):
        W3 = lin_w.reshape(H, C, HID)
        As = jnp.einsum("hck,hc->kh", W3, a_s)
        Ad = jnp.einsum("hck,hc->kh", W3, a_d)
        B3 = lew.reshape(H, C, HID)
        Bm = jnp.einsum("hck,hc->kh", B3, a_e)
        u = W_ee[:, 0] @ Bm
        v = b_ee @ Bm
        return As, Ad, jnp.tile(u, H), jnp.tile(v, H)

    bs = 1.0 / jnp.sqrt(1.0 + 1e-5)
    As0, Ad0, ut0, vt0 = prep(lin_w0, att_src0, att_dst0, att_edge0, lin_edge_w0)
    As1, Ad1, ut1, vt1 = prep(lin_w1, att_src1, att_dst1, att_edge1, lin_edge_w1)
    sc0 = (bs * bn_g0).reshape(1, HID)
    bi0 = (conv_b0 * bs * bn_g0 + bn_b0).reshape(1, HID)
    sc1 = (bs * bn_g1).reshape(1, HID)
    bi1 = (conv_b1 * bs * bn_g1 + bn_b1).reshape(1, HID)

    zn = jnp.zeros((784, HID), f32)
    zd = jnp.zeros((784 * H,), f32)
    ninf = jnp.full(((R2 - NQ) * H,), -jnp.inf, f32)

    h0, xl0, asrc0, adst0 = pl.pallas_call(
        _enc_dense_body,
        grid=(_NB,),
        in_specs=[
            pl.BlockSpec((_BN, IN), lambda i: (i, 0)),
            pl.BlockSpec((IN, HID), lambda i: (0, 0)),
            pl.BlockSpec((1, HID), lambda i: (0, 0)),
            pl.BlockSpec((HID, HID), lambda i: (0, 0)),
            pl.BlockSpec((HID, H), lambda i: (0, 0)),
            pl.BlockSpec((HID, H), lambda i: (0, 0)),
        ],
        out_specs=[
            pl.BlockSpec((_BN, HID), lambda i: (i, 0)),
            pl.BlockSpec((_BN, HID), lambda i: (i, 0)),
            pl.BlockSpec((_BN, H), lambda i: (i, 0)),
            pl.BlockSpec((_BN, H), lambda i: (i, 0)),
        ],
        out_shape=[
            jax.ShapeDtypeStruct((N, HID), f32),
            jax.ShapeDtypeStruct((N, HID), f32),
            jax.ShapeDtypeStruct((N, H), f32),
            jax.ShapeDtypeStruct((N, H), f32),
        ],
    )(x, W_ne.T, b_ne.reshape(1, HID), lin_w0.T, As0, Ad0)

    num0, den0 = _edge_kernel()(src, dst, eaf,
                                asrc0.reshape(N * H), adst0.reshape(N * H),
                                xl0, ut0, vt0, zn, zd, ninf)
    den0 = den0.reshape(N, H)

    h1, xl1, asrc1, adst1 = pl.pallas_call(
        _epi_dense_body,
        grid=(_NB,),
        in_specs=[
            pl.BlockSpec((_BN, HID), lambda i: (i, 0)),
            pl.BlockSpec((_BN, H), lambda i: (i, 0)),
            pl.BlockSpec((_BN, HID), lambda i: (i, 0)),
            pl.BlockSpec((1, HID), lambda i: (0, 0)),
            pl.BlockSpec((1, HID), lambda i: (0, 0)),
            pl.BlockSpec((HID, HID), lambda i: (0, 0)),
            pl.BlockSpec((HID, H), lambda i: (0, 0)),
            pl.BlockSpec((HID, H), lambda i: (0, 0)),
        ],
        out_specs=[
            pl.BlockSpec((_BN, HID), lambda i: (i, 0)),
            pl.BlockSpec((_BN, HID), lambda i: (i, 0)),
            pl.BlockSpec((_BN, H), lambda i: (i, 0)),
            pl.BlockSpec((_BN, H), lambda i: (i, 0)),
        ],
        out_shape=[
            jax.ShapeDtypeStruct((N, HID), f32),
            jax.ShapeDtypeStruct((N, HID), f32),
            jax.ShapeDtypeStruct((N, H), f32),
            jax.ShapeDtypeStruct((N, H), f32),
        ],
    )(num0, den0, h0, sc0, bi0, lin_w1.T, As1, Ad1)

    num1, den1 = _edge_kernel()(src, dst, eaf,
                                asrc1.reshape(N * H), adst1.reshape(N * H),
                                xl1, ut1, vt1, zn, zd, ninf)
    den1 = den1.reshape(N, H)

    out, _sums, _cnt = pl.pallas_call(
        _final_body,
        grid=(_NB,),
        in_specs=[
            pl.BlockSpec((_BN, HID), lambda i: (i, 0)),
            pl.BlockSpec((_BN, H), lambda i: (i, 0)),
            pl.BlockSpec((_BN, HID), lambda i: (i, 0)),
            pl.BlockSpec((1, HID), lambda i: (0, 0)),
            pl.BlockSpec((1, HID), lambda i: (0, 0)),
            pl.BlockSpec((1, 1, _BN), lambda i: (i, 0, 0)),
            pl.BlockSpec((HID, HID // 2), lambda i: (0, 0)),
            pl.BlockSpec((1, HID // 2), lambda i: (0, 0)),
            pl.BlockSpec((HID // 2, 1), lambda i: (0, 0)),
            pl.BlockSpec((1, 1), lambda i: (0, 0)),
        ],
        out_specs=[
            pl.BlockSpec((1, G), lambda i: (0, 0)),
            pl.BlockSpec((G, HID), lambda i: (0, 0)),
            pl.BlockSpec((G, HID), lambda i: (0, 0)),
        ],
        out_shape=[
            jax.ShapeDtypeStruct((1, G), f32),
            jax.ShapeDtypeStruct((G, HID), f32),
            jax.ShapeDtypeStruct((G, HID), f32),
        ],
    )(num1, den1, h1, sc1, bi1, batch.reshape(_NB, 1, _BN),
      W_c1.T, b_c1.reshape(1, HID // 2), W_c2.T, b_c2.reshape(1, 1))

    return out.reshape(G)


# speedup vs baseline: 69.6647x; 1.2601x over previous
import functools

import jax
import jax.numpy as jnp
from jax import lax
from jax.experimental import pallas as pl
from jax.experimental.pallas import tpu as pltpu
from jax.experimental.pallas import tpu_sc as plsc

N = 50000
E = 800000
IN = 128
HID = 64
H = 4
C = 16
G = 64

NSUB = 16
NCORE = 2
N2 = N // 2
NQ = N // 4
R2 = 12544
SENTQ = NQ
EPW = E // NSUB
K = 400
KP = 512
NCH = EPW // K

_NB = 125
_BN = 400


def _enc_dense_body(x_ref, wne_ref, bne_ref, lwT_ref, as_ref, ad_ref,
                    h_ref, xl_ref, asrc_ref, adst_ref):
    h = jnp.dot(x_ref[...], wne_ref[...], preferred_element_type=jnp.float32)
    h = h + bne_ref[...]
    h_ref[...] = h
    xl_ref[...] = jnp.dot(h, lwT_ref[...], preferred_element_type=jnp.float32)
    asrc_ref[...] = jnp.dot(h, as_ref[...], preferred_element_type=jnp.float32)
    adst_ref[...] = jnp.dot(h, ad_ref[...], preferred_element_type=jnp.float32)


def _epi_dense_body(num_ref, den_ref, hin_ref, sc_ref, bi_ref,
                    lwT_ref, as_ref, ad_ref,
                    h_ref, xl_ref, asrc_ref, adst_ref):
    heads = lax.broadcasted_iota(jnp.int32, (H, HID), 1) // C
    R = (heads == lax.broadcasted_iota(jnp.int32, (H, HID), 0)).astype(jnp.float32)
    denx = jnp.dot(den_ref[...], R, preferred_element_type=jnp.float32)
    gat = num_ref[...] / (denx + 1e-16)
    h2 = gat * sc_ref[...] + bi_ref[...]
    h = jnp.maximum(h2, 0.0) + hin_ref[...]
    h_ref[...] = h
    xl_ref[...] = jnp.dot(h, lwT_ref[...], preferred_element_type=jnp.float32)
    asrc_ref[...] = jnp.dot(h, as_ref[...], preferred_element_type=jnp.float32)
    adst_ref[...] = jnp.dot(h, ad_ref[...], preferred_element_type=jnp.float32)


def _final_body(num_ref, den_ref, hin_ref, sc_ref, bi_ref, batch_ref,
                wc1_ref, bc1_ref, wc2_ref, bc2_ref,
                out_ref, sums_ref, cnt_ref):
    i = pl.program_id(0)
    heads = lax.broadcasted_iota(jnp.int32, (H, HID), 1) // C
    R = (heads == lax.broadcasted_iota(jnp.int32, (H, HID), 0)).astype(jnp.float32)
    denx = jnp.dot(den_ref[...], R, preferred_element_type=jnp.float32)
    gat = num_ref[...] / (denx + 1e-16)
    h2 = gat * sc_ref[...] + bi_ref[...]
    h = jnp.maximum(h2, 0.0) + hin_ref[...]

    b = batch_ref[0, 0, :]
    oh = (b[:, None] == lax.broadcasted_iota(jnp.int32, (_BN, G), 1)).astype(jnp.float32)
    sums_d = lax.dot_general(oh, h, (((0,), (0,)), ((), ())),
                             preferred_element_type=jnp.float32)
    cnt_d = lax.dot_general(oh, jnp.ones((_BN, HID), jnp.float32),
                            (((0,), (0,)), ((), ())),
                            preferred_element_type=jnp.float32)

    @pl.when(i == 0)
    def _():
        sums_ref[...] = sums_d
        cnt_ref[...] = cnt_d

    @pl.when(i > 0)
    def _():
        sums_ref[...] += sums_d
        cnt_ref[...] += cnt_d

    @pl.when(i == _NB - 1)
    def _():
        pooled = sums_ref[...] / jnp.maximum(cnt_ref[...], 1.0)
        hc = jnp.dot(pooled, wc1_ref[...], preferred_element_type=jnp.float32)
        hc = jnp.maximum(hc + bc1_ref[...], 0.0)
        o = jnp.dot(hc, wc2_ref[...], preferred_element_type=jnp.float32)
        o = jax.nn.sigmoid(o + bc2_ref[...])
        out_ref[...] = o.T


KR = K + 16


def _edge_body(sde_h, asrc_h, adst_h, xl_h, ut_h, vt_h,
               zn_h, zd_h, ninf_h,
               num_h, den_h,
               S_asrc, S_adst, S_num, S_den,
               sdeA, sdeB, srcC, locC, eaC,
               idxs_v, idxa_v, idxd_v, idxe_v,
               ag_v, bg_v, ex_v, xlG, uv_v, vv_v,
               sem0, sem1, sem2, semLA, semLB):
    cid = lax.axis_index("c")
    sid = lax.axis_index("s")

    @pl.when(sid < NSUB - 1)
    def _():
        pltpu.sync_copy(asrc_h.at[pl.ds(sid * 12512, 12512)],
                        S_asrc.at[pl.ds(sid * 12512, 12512)])

    @pl.when(sid == NSUB - 1)
    def _():
        pltpu.sync_copy(asrc_h.at[pl.ds(15 * 12512, 12320)],
                        S_asrc.at[pl.ds(15 * 12512, 12320)])

    pltpu.sync_copy(ut_h, uv_v)
    pltpu.sync_copy(vt_h, vv_v)

    utile = uv_v[...]
    vtile = vv_v[...]
    expand4 = lax.iota(jnp.int32, 16) // jnp.int32(4)
    headpat = lax.iota(jnp.int32, 16) % jnp.int32(4)
    lanes16 = lax.iota(jnp.int32, 16)

    def drain():
        def build(io, c2):
            sv = srcC[pl.ds(io * 16, 16)]
            lv = locC[pl.ds(io * 16, 16)]
            idxs_v[pl.ds(io * 16, 16)] = sv
            idxd_v[pl.ds(io * 16, 16)] = lv
            for q in range(4):
                idxc = expand4 + jnp.int32(4 * q)
                sexp = jnp.take_along_axis(sv, idxc, axis=0,
                                           mode="promise_in_bounds")
                lexp = jnp.take_along_axis(lv, idxc, axis=0,
                                           mode="promise_in_bounds")
                j0 = io * 64 + q * 16
                idxa_v[pl.ds(j0, 16)] = sexp * H + headpat
                idxe_v[pl.ds(j0, 16)] = lexp * H + headpat
            return c2
        lax.fori_loop(0, K // 16, build, 0)

        cp0 = pltpu.async_copy(S_asrc.at[idxa_v], ag_v, sem0)
        cp1 = pltpu.async_copy(S_adst.at[idxe_v], bg_v, sem1)
        cp2 = pltpu.async_copy(xl_h.at[idxs_v], xlG, sem2)
        cp0.wait()
        cp1.wait()
        cp2.wait()

        def exloop(io, c2):
            eav = eaC[pl.ds(io * 16, 16)]
            for q in range(4):
                idxc = expand4 + jnp.int32(4 * q)
                aeq = jnp.take_along_axis(eav, idxc, axis=0,
                                          mode="promise_in_bounds")
                aeq = aeq * utile + vtile
                j0 = io * 64 + q * 16
                a = ag_v[pl.ds(j0, 16)] + bg_v[pl.ds(j0, 16)] + aeq
                a = jnp.maximum(a, a * 0.2)
                ex_v[pl.ds(j0, 16)] = jnp.exp(a)
            return c2
        lax.fori_loop(0, K // 16, exloop, 0)

        def mulloop(io, c2):
            for q in range(4):
                exq = ex_v[pl.ds(io * 64 + q * 16, 16)]
                for t in range(4):
                    e = io * 16 + q * 4 + t
                    for hh in range(H):
                        splat = jnp.take_along_axis(
                            exq, jnp.full((16,), t * 4 + hh, jnp.int32),
                            axis=0, mode="promise_in_bounds")
                        v = xlG[e, pl.ds(hh * 16, 16)]
                        xlG[e, pl.ds(hh * 16, 16)] = v * splat
            return c2
        lax.fori_loop(0, K // 16, mulloop, 0)

        pltpu.sync_copy(ex_v, S_den.at[idxe_v], add=True)
        pltpu.sync_copy(xlG, S_num.at[idxd_v], add=True)

    for p in range(2):
        qb = cid * N2 + p * NQ

        @pl.when(sid < NSUB - 1)
        def _():
            pltpu.sync_copy(adst_h.at[pl.ds(qb * H + sid * 3136, 3136)],
                            S_adst.at[pl.ds(sid * 3136, 3136)])

        @pl.when(sid == NSUB - 1)
        def _():
            pltpu.sync_copy(adst_h.at[pl.ds(qb * H + 15 * 3136, 2960)],
                            S_adst.at[pl.ds(15 * 3136, 2960)])

        @pl.when(sid == 0)
        def _():
            pltpu.sync_copy(ninf_h, S_adst.at[pl.ds(NQ * H, (R2 - NQ) * H)])

        pltpu.sync_copy(zn_h, S_num.at[pl.ds(sid * 784, 784)])
        pltpu.sync_copy(zd_h, S_den.at[pl.ds(sid * 3136, 3136)])

        plsc.subcore_barrier()

        qbv = jnp.full((16,), qb, jnp.int32)

        def fire_load(j, buf, sem):
            base = pl.multiple_of(sid * EPW + j * K, 8)
            return pltpu.async_copy(sde_h.at[:, pl.ds(base, K)], buf, sem)

        def wait_load(j, buf, sem):
            base = pl.multiple_of(sid * EPW + j * K, 8)
            pltpu.make_async_copy(sde_h.at[:, pl.ds(base, K)], buf, sem).wait()

        def route_chunk(buf, fill):
            def route(io, f):
                sv = buf[0, pl.ds(io * 16, 16)]
                d = buf[1, pl.ds(io * 16, 16)]
                eav = plsc.bitcast(buf[2, pl.ds(io * 16, 16)], jnp.float32)
                own = (d >= qbv) & (d < qbv + jnp.int32(NQ))
                loc = d - qbv
                plsc.store_compressed(srcC.at[pl.ds(f, 16)], sv, mask=own)
                plsc.store_compressed(locC.at[pl.ds(f, 16)], loc, mask=own)
                plsc.store_compressed(eaC.at[pl.ds(f, 16)], eav, mask=own)
                cnt = plsc.all_reduce_population_count(own)
                f = f + (cnt[0] if getattr(cnt, "ndim", 0) else cnt)

                def do_drain(ff):
                    drain()
                    srcC[pl.ds(0, 16)] = srcC[pl.ds(K, 16)]
                    locC[pl.ds(0, 16)] = locC[pl.ds(K, 16)]
                    eaC[pl.ds(0, 16)] = eaC[pl.ds(K, 16)]
                    return ff - jnp.int32(K)

                return lax.cond(f >= K, do_drain, lambda ff: ff, f)
            return lax.fori_loop(0, K // 16, route, fill)

        fire_load(0, sdeA, semLA)

        def chunk2(j2, fill):
            wait_load(2 * j2, sdeA, semLA)
            fire_load(2 * j2 + 1, sdeB, semLB)
            fill = route_chunk(sdeA, fill)
            wait_load(2 * j2 + 1, sdeB, semLB)
            fire_load(2 * j2 + 2, sdeA, semLA)
            return route_chunk(sdeB, fill)

        fill = lax.fori_loop(0, NCH // 2, chunk2, jnp.int32(0))
        wait_load(NCH - 1, sdeA, semLA)
        fill = route_chunk(sdeA, fill)

        fillv = jnp.full((16,), 0, jnp.int32) + fill

        def pad(io, c2):
            pos = lanes16 + jnp.int32(io * 16)
            m = pos >= fillv
            sv = srcC[pl.ds(io * 16, 16)]
            lv = locC[pl.ds(io * 16, 16)]
            ev = eaC[pl.ds(io * 16, 16)]
            srcC[pl.ds(io * 16, 16)] = jnp.where(m, jnp.zeros((16,), jnp.int32), sv)
            locC[pl.ds(io * 16, 16)] = jnp.where(
                m, jnp.full((16,), SENTQ, jnp.int32), lv)
            eaC[pl.ds(io * 16, 16)] = jnp.where(m, jnp.zeros((16,), jnp.float32), ev)
            return c2
        lax.fori_loop(0, K // 16, pad, 0)
        drain()

        plsc.subcore_barrier()

        @pl.when(sid < NSUB - 1)
        def _():
            pltpu.sync_copy(S_num.at[pl.ds(sid * 784, 784)],
                            num_h.at[pl.ds(qb + sid * 784, 784)])
            pltpu.sync_copy(S_den.at[pl.ds(sid * 3136, 3136)],
                            den_h.at[pl.ds(qb * H + sid * 3136, 3136)])

        @pl.when(sid == NSUB - 1)
        def _():
            pltpu.sync_copy(S_num.at[pl.ds(15 * 784, 740)],
                            num_h.at[pl.ds(qb + 15 * 784, 740)])
            pltpu.sync_copy(S_den.at[pl.ds(15 * 3136, 2960)],
                            den_h.at[pl.ds(qb * H + 15 * 3136, 2960)])

        plsc.subcore_barrier()


_edge_kernel = functools.partial(
    pl.kernel,
    _edge_body,
    out_type=(jax.ShapeDtypeStruct((N, HID), jnp.float32),
              jax.ShapeDtypeStruct((N * H,), jnp.float32)),
    mesh=plsc.VectorSubcoreMesh(core_axis_name="c", subcore_axis_name="s"),
    compiler_params=pltpu.CompilerParams(needs_layout_passes=False,
                                         use_tc_tiling_on_sc=False),
    scratch_types=(
        pltpu.VMEM_SHARED((N * H,), jnp.float32),
        pltpu.VMEM_SHARED((R2 * H,), jnp.float32),
        pltpu.VMEM_SHARED((R2, HID), jnp.float32),
        pltpu.VMEM_SHARED((R2 * H,), jnp.float32),
        pltpu.VMEM((3, K), jnp.int32),
        pltpu.VMEM((3, K), jnp.int32),
        pltpu.VMEM((KR,), jnp.int32),
        pltpu.VMEM((KR,), jnp.int32),
        pltpu.VMEM((KR,), jnp.float32),
        pltpu.VMEM((K,), jnp.int32),
        pltpu.VMEM((K * H,), jnp.int32),
        pltpu.VMEM((K,), jnp.int32),
        pltpu.VMEM((K * H,), jnp.int32),
        pltpu.VMEM((K * H,), jnp.float32),
        pltpu.VMEM((K * H,), jnp.float32),
        pltpu.VMEM((K * H,), jnp.float32),
        pltpu.VMEM((K, HID), jnp.float32),
        pltpu.VMEM((16,), jnp.float32),
        pltpu.VMEM((16,), jnp.float32),
        pltpu.SemaphoreType.DMA,
        pltpu.SemaphoreType.DMA,
        pltpu.SemaphoreType.DMA,
        pltpu.SemaphoreType.DMA,
        pltpu.SemaphoreType.DMA,
    ),
)


def kernel(x, edge_index, edge_attr, batch, W_ne, b_ne, W_ee, b_ee, lin_w0, att_src0, att_dst0, att_edge0, lin_edge_w0, conv_b0, bn_g0, bn_b0, lin_w1, att_src1, att_dst1, att_edge1, lin_edge_w1, conv_b1, bn_g1, bn_b1, W_c1, b_c1, W_c2, b_c2):
    f32 = jnp.float32
    eaf = edge_attr.reshape(E)
    sde = jnp.concatenate(
        [edge_index,
         lax.bitcast_convert_type(eaf, jnp.int32).reshape(1, E)], axis=0)

    def prep(lin_w, a_s, a_d, a_e, lew):
        W3 = lin_w.reshape(H, C, HID)
        As = jnp.einsum("hck,hc->kh", W3, a_s)
        Ad = jnp.einsum("hck,hc->kh", W3, a_d)
        B3 = lew.reshape(H, C, HID)
        Bm = jnp.einsum("hck,hc->kh", B3, a_e)
        u = W_ee[:, 0] @ Bm
        v = b_ee @ Bm
        return As, Ad, jnp.tile(u, H), jnp.tile(v, H)

    bs = 1.0 / jnp.sqrt(1.0 + 1e-5)
    As0, Ad0, ut0, vt0 = prep(lin_w0, att_src0, att_dst0, att_edge0, lin_edge_w0)
    As1, Ad1, ut1, vt1 = prep(lin_w1, att_src1, att_dst1, att_edge1, lin_edge_w1)
    sc0 = (bs * bn_g0).reshape(1, HID)
    bi0 = (conv_b0 * bs * bn_g0 + bn_b0).reshape(1, HID)
    sc1 = (bs * bn_g1).reshape(1, HID)
    bi1 = (conv_b1 * bs * bn_g1 + bn_b1).reshape(1, HID)

    zn = jnp.zeros((784, HID), f32)
    zd = jnp.zeros((784 * H,), f32)
    ninf = jnp.full(((R2 - NQ) * H,), -jnp.inf, f32)

    h0, xl0, asrc0, adst0 = pl.pallas_call(
        _enc_dense_body,
        grid=(_NB,),
        in_specs=[
            pl.BlockSpec((_BN, IN), lambda i: (i, 0)),
            pl.BlockSpec((IN, HID), lambda i: (0, 0)),
            pl.BlockSpec((1, HID), lambda i: (0, 0)),
            pl.BlockSpec((HID, HID), lambda i: (0, 0)),
            pl.BlockSpec((HID, H), lambda i: (0, 0)),
            pl.BlockSpec((HID, H), lambda i: (0, 0)),
        ],
        out_specs=[
            pl.BlockSpec((_BN, HID), lambda i: (i, 0)),
            pl.BlockSpec((_BN, HID), lambda i: (i, 0)),
            pl.BlockSpec((_BN, H), lambda i: (i, 0)),
            pl.BlockSpec((_BN, H), lambda i: (i, 0)),
        ],
        out_shape=[
            jax.ShapeDtypeStruct((N, HID), f32),
            jax.ShapeDtypeStruct((N, HID), f32),
            jax.ShapeDtypeStruct((N, H), f32),
            jax.ShapeDtypeStruct((N, H), f32),
        ],
    )(x, W_ne.T, b_ne.reshape(1, HID), lin_w0.T, As0, Ad0)

    num0, den0 = _edge_kernel()(sde,
                                asrc0.reshape(N * H), adst0.reshape(N * H),
                                xl0, ut0, vt0, zn, zd, ninf)
    den0 = den0.reshape(N, H)

    h1, xl1, asrc1, adst1 = pl.pallas_call(
        _epi_dense_body,
        grid=(_NB,),
        in_specs=[
            pl.BlockSpec((_BN, HID), lambda i: (i, 0)),
            pl.BlockSpec((_BN, H), lambda i: (i, 0)),
            pl.BlockSpec((_BN, HID), lambda i: (i, 0)),
            pl.BlockSpec((1, HID), lambda i: (0, 0)),
            pl.BlockSpec((1, HID), lambda i: (0, 0)),
            pl.BlockSpec((HID, HID), lambda i: (0, 0)),
            pl.BlockSpec((HID, H), lambda i: (0, 0)),
            pl.BlockSpec((HID, H), lambda i: (0, 0)),
        ],
        out_specs=[
            pl.BlockSpec((_BN, HID), lambda i: (i, 0)),
            pl.BlockSpec((_BN, HID), lambda i: (i, 0)),
            pl.BlockSpec((_BN, H), lambda i: (i, 0)),
            pl.BlockSpec((_BN, H), lambda i: (i, 0)),
        ],
        out_shape=[
            jax.ShapeDtypeStruct((N, HID), f32),
            jax.ShapeDtypeStruct((N, HID), f32),
            jax.ShapeDtypeStruct((N, H), f32),
            jax.ShapeDtypeStruct((N, H), f32),
        ],
    )(num0, den0, h0, sc0, bi0, lin_w1.T, As1, Ad1)

    num1, den1 = _edge_kernel()(sde,
                                asrc1.reshape(N * H), adst1.reshape(N * H),
                                xl1, ut1, vt1, zn, zd, ninf)
    den1 = den1.reshape(N, H)

    out, _sums, _cnt = pl.pallas_call(
        _final_body,
        grid=(_NB,),
        in_specs=[
            pl.BlockSpec((_BN, HID), lambda i: (i, 0)),
            pl.BlockSpec((_BN, H), lambda i: (i, 0)),
            pl.BlockSpec((_BN, HID), lambda i: (i, 0)),
            pl.BlockSpec((1, HID), lambda i: (0, 0)),
            pl.BlockSpec((1, HID), lambda i: (0, 0)),
            pl.BlockSpec((1, 1, _BN), lambda i: (i, 0, 0)),
            pl.BlockSpec((HID, HID // 2), lambda i: (0, 0)),
            pl.BlockSpec((1, HID // 2), lambda i: (0, 0)),
            pl.BlockSpec((HID // 2, 1), lambda i: (0, 0)),
            pl.BlockSpec((1, 1), lambda i: (0, 0)),
        ],
        out_specs=[
            pl.BlockSpec((1, G), lambda i: (0, 0)),
            pl.BlockSpec((G, HID), lambda i: (0, 0)),
            pl.BlockSpec((G, HID), lambda i: (0, 0)),
        ],
        out_shape=[
            jax.ShapeDtypeStruct((1, G), f32),
            jax.ShapeDtypeStruct((G, HID), f32),
            jax.ShapeDtypeStruct((G, HID), f32),
        ],
    )(num1, den1, h1, sc1, bi1, batch.reshape(_NB, 1, _BN),
      W_c1.T, b_c1.reshape(1, HID // 2), W_c2.T, b_c2.reshape(1, 1))

    return out.reshape(G)


# speedup vs baseline: 91.9882x; 1.3204x over previous
import functools

import jax
import jax.numpy as jnp
from jax import lax
from jax.experimental import pallas as pl
from jax.experimental.pallas import tpu as pltpu
from jax.experimental.pallas import tpu_sc as plsc

N = 50000
E = 800000
IN = 128
HID = 64
H = 4
C = 16
G = 64

NSUB = 16
NCORE = 2
N2 = N // 2
NQ = N // 4
R2 = 12544
SENTQ = NQ
EPW = E // NSUB
K = 400
KP = 512
NCH = EPW // K

_NB = 125
_BN = 400


def _enc_dense_body(x_ref, wne_ref, bne_ref, lwT_ref, as_ref, ad_ref,
                    h_ref, xl_ref, asrc_ref, adst_ref):
    h = jnp.dot(x_ref[...], wne_ref[...], preferred_element_type=jnp.float32)
    h = h + bne_ref[...]
    h_ref[...] = h
    xl_ref[...] = jnp.dot(h, lwT_ref[...], preferred_element_type=jnp.float32)
    asrc_ref[...] = jnp.dot(h, as_ref[...], preferred_element_type=jnp.float32)
    adst_ref[...] = jnp.dot(h, ad_ref[...], preferred_element_type=jnp.float32)


def _epi_dense_body(num_ref, den_ref, hin_ref, sc_ref, bi_ref,
                    lwT_ref, as_ref, ad_ref,
                    h_ref, xl_ref, asrc_ref, adst_ref):
    heads = lax.broadcasted_iota(jnp.int32, (H, HID), 1) // C
    R = (heads == lax.broadcasted_iota(jnp.int32, (H, HID), 0)).astype(jnp.float32)
    denx = jnp.dot(den_ref[...], R, preferred_element_type=jnp.float32)
    gat = num_ref[...] / (denx + 1e-16)
    h2 = gat * sc_ref[...] + bi_ref[...]
    h = jnp.maximum(h2, 0.0) + hin_ref[...]
    h_ref[...] = h
    xl_ref[...] = jnp.dot(h, lwT_ref[...], preferred_element_type=jnp.float32)
    asrc_ref[...] = jnp.dot(h, as_ref[...], preferred_element_type=jnp.float32)
    adst_ref[...] = jnp.dot(h, ad_ref[...], preferred_element_type=jnp.float32)


def _final_body(num_ref, den_ref, hin_ref, sc_ref, bi_ref, batch_ref,
                wc1_ref, bc1_ref, wc2_ref, bc2_ref,
                out_ref, sums_ref, cnt_ref):
    i = pl.program_id(0)
    heads = lax.broadcasted_iota(jnp.int32, (H, HID), 1) // C
    R = (heads == lax.broadcasted_iota(jnp.int32, (H, HID), 0)).astype(jnp.float32)
    denx = jnp.dot(den_ref[...], R, preferred_element_type=jnp.float32)
    gat = num_ref[...] / (denx + 1e-16)
    h2 = gat * sc_ref[...] + bi_ref[...]
    h = jnp.maximum(h2, 0.0) + hin_ref[...]

    b = batch_ref[0, 0, :]
    oh = (b[:, None] == lax.broadcasted_iota(jnp.int32, (_BN, G), 1)).astype(jnp.float32)
    sums_d = lax.dot_general(oh, h, (((0,), (0,)), ((), ())),
                             preferred_element_type=jnp.float32)
    cnt_d = lax.dot_general(oh, jnp.ones((_BN, HID), jnp.float32),
                            (((0,), (0,)), ((), ())),
                            preferred_element_type=jnp.float32)

    @pl.when(i == 0)
    def _():
        sums_ref[...] = sums_d
        cnt_ref[...] = cnt_d

    @pl.when(i > 0)
    def _():
        sums_ref[...] += sums_d
        cnt_ref[...] += cnt_d

    @pl.when(i == _NB - 1)
    def _():
        pooled = sums_ref[...] / jnp.maximum(cnt_ref[...], 1.0)
        hc = jnp.dot(pooled, wc1_ref[...], preferred_element_type=jnp.float32)
        hc = jnp.maximum(hc + bc1_ref[...], 0.0)
        o = jnp.dot(hc, wc2_ref[...], preferred_element_type=jnp.float32)
        o = jax.nn.sigmoid(o + bc2_ref[...])
        out_ref[...] = o.T


KR = K + 16


def _edge_body(sde_h, asrc_h, adst_h, xl_h, ut_h, vt_h,
               zn_h, zd_h, ninf_h,
               num_h, den_h,
               S_asrc, S_adst, S_num, S_den,
               sdeA, sdeB, srcC, locC, eaC,
               idxs_v, idxa_v, idxd_v, idxe_v,
               ag_v, bg_v, ex_v, xlG, uv_v, vv_v,
               sem0, sem1, sem2, semLA, semLB, semS1, semS2):
    cid = lax.axis_index("c")
    sid = lax.axis_index("s")

    @pl.when(sid < NSUB - 1)
    def _():
        pltpu.sync_copy(asrc_h.at[pl.ds(sid * 12512, 12512)],
                        S_asrc.at[pl.ds(sid * 12512, 12512)])

    @pl.when(sid == NSUB - 1)
    def _():
        pltpu.sync_copy(asrc_h.at[pl.ds(15 * 12512, 12320)],
                        S_asrc.at[pl.ds(15 * 12512, 12320)])

    pltpu.sync_copy(ut_h, uv_v)
    pltpu.sync_copy(vt_h, vv_v)

    utile = uv_v[...]
    vtile = vv_v[...]
    expand4 = lax.iota(jnp.int32, 16) // jnp.int32(4)
    headpat = lax.iota(jnp.int32, 16) % jnp.int32(4)
    lanes16 = lax.iota(jnp.int32, 16)

    def wait_scatters():
        pltpu.make_async_copy(ex_v, S_den.at[idxe_v], semS1).wait()
        pltpu.make_async_copy(xlG, S_num.at[idxd_v], semS2).wait()

    def drain(primed):
        lax.cond(primed != 0, wait_scatters, lambda: None)

        def build(io, c2):
            sv = srcC[pl.ds(io * 16, 16)]
            lv = locC[pl.ds(io * 16, 16)]
            idxs_v[pl.ds(io * 16, 16)] = sv
            idxd_v[pl.ds(io * 16, 16)] = lv
            for q in range(4):
                idxc = expand4 + jnp.int32(4 * q)
                sexp = jnp.take_along_axis(sv, idxc, axis=0,
                                           mode="promise_in_bounds")
                lexp = jnp.take_along_axis(lv, idxc, axis=0,
                                           mode="promise_in_bounds")
                j0 = io * 64 + q * 16
                idxa_v[pl.ds(j0, 16)] = sexp * H + headpat
                idxe_v[pl.ds(j0, 16)] = lexp * H + headpat
            return c2
        lax.fori_loop(0, K // 16, build, 0)

        cp0 = pltpu.async_copy(S_asrc.at[idxa_v], ag_v, sem0)
        cp1 = pltpu.async_copy(S_adst.at[idxe_v], bg_v, sem1)
        cp2 = pltpu.async_copy(xl_h.at[idxs_v], xlG, sem2)
        cp0.wait()
        cp1.wait()
        cp2.wait()

        def exloop(io, c2):
            eav = eaC[pl.ds(io * 16, 16)]
            for q in range(4):
                idxc = expand4 + jnp.int32(4 * q)
                aeq = jnp.take_along_axis(eav, idxc, axis=0,
                                          mode="promise_in_bounds")
                aeq = aeq * utile + vtile
                j0 = io * 64 + q * 16
                a = ag_v[pl.ds(j0, 16)] + bg_v[pl.ds(j0, 16)] + aeq
                a = jnp.maximum(a, a * 0.2)
                ex_v[pl.ds(j0, 16)] = jnp.exp(a)
            return c2
        lax.fori_loop(0, K // 16, exloop, 0, unroll=2)

        def mulloop(io, c2):
            for q in range(4):
                exq = ex_v[pl.ds(io * 64 + q * 16, 16)]
                for t in range(4):
                    e = io * 16 + q * 4 + t
                    for hh in range(H):
                        splat = jnp.take_along_axis(
                            exq, jnp.full((16,), t * 4 + hh, jnp.int32),
                            axis=0, mode="promise_in_bounds")
                        v = xlG[e, pl.ds(hh * 16, 16)]
                        xlG[e, pl.ds(hh * 16, 16)] = v * splat
            return c2
        lax.fori_loop(0, K // 16, mulloop, 0, unroll=2)

        pltpu.async_copy(ex_v, S_den.at[idxe_v], semS1, add=True)
        pltpu.async_copy(xlG, S_num.at[idxd_v], semS2, add=True)

    for p in range(2):
        qb = cid * N2 + p * NQ

        @pl.when(sid < NSUB - 1)
        def _():
            pltpu.sync_copy(adst_h.at[pl.ds(qb * H + sid * 3136, 3136)],
                            S_adst.at[pl.ds(sid * 3136, 3136)])

        @pl.when(sid == NSUB - 1)
        def _():
            pltpu.sync_copy(adst_h.at[pl.ds(qb * H + 15 * 3136, 2960)],
                            S_adst.at[pl.ds(15 * 3136, 2960)])

        @pl.when(sid == 0)
        def _():
            pltpu.sync_copy(ninf_h, S_adst.at[pl.ds(NQ * H, (R2 - NQ) * H)])

        pltpu.sync_copy(zn_h, S_num.at[pl.ds(sid * 784, 784)])
        pltpu.sync_copy(zd_h, S_den.at[pl.ds(sid * 3136, 3136)])

        plsc.subcore_barrier()

        qbv = jnp.full((16,), qb, jnp.int32)

        def fire_load(j, buf, sem):
            base = pl.multiple_of(sid * EPW + j * K, 8)
            return pltpu.async_copy(sde_h.at[:, pl.ds(base, K)], buf, sem)

        def wait_load(j, buf, sem):
            base = pl.multiple_of(sid * EPW + j * K, 8)
            pltpu.make_async_copy(sde_h.at[:, pl.ds(base, K)], buf, sem).wait()

        def route_chunk(buf, carry):
            def route(io, carry):
                f, primed = carry
                sv = buf[0, pl.ds(io * 16, 16)]
                d = buf[1, pl.ds(io * 16, 16)]
                eav = plsc.bitcast(buf[2, pl.ds(io * 16, 16)], jnp.float32)
                own = (d >= qbv) & (d < qbv + jnp.int32(NQ))
                loc = d - qbv
                plsc.store_compressed(srcC.at[pl.ds(f, 16)], sv, mask=own)
                plsc.store_compressed(locC.at[pl.ds(f, 16)], loc, mask=own)
                plsc.store_compressed(eaC.at[pl.ds(f, 16)], eav, mask=own)
                cnt = plsc.all_reduce_population_count(own)
                f = f + (cnt[0] if getattr(cnt, "ndim", 0) else cnt)

                def do_drain(ff, pr):
                    drain(pr)
                    srcC[pl.ds(0, 16)] = srcC[pl.ds(K, 16)]
                    locC[pl.ds(0, 16)] = locC[pl.ds(K, 16)]
                    eaC[pl.ds(0, 16)] = eaC[pl.ds(K, 16)]
                    return ff - jnp.int32(K), jnp.int32(1)

                return lax.cond(f >= K, do_drain,
                                lambda ff, pr: (ff, pr), f, primed)
            return lax.fori_loop(0, K // 16, route, carry)

        fire_load(0, sdeA, semLA)

        def chunk2(j2, carry):
            wait_load(2 * j2, sdeA, semLA)
            fire_load(2 * j2 + 1, sdeB, semLB)
            carry = route_chunk(sdeA, carry)
            wait_load(2 * j2 + 1, sdeB, semLB)
            fire_load(2 * j2 + 2, sdeA, semLA)
            return route_chunk(sdeB, carry)

        fill, primed = lax.fori_loop(0, NCH // 2, chunk2,
                                     (jnp.int32(0), jnp.int32(0)))
        wait_load(NCH - 1, sdeA, semLA)
        fill, primed = route_chunk(sdeA, (fill, primed))

        fillv = jnp.full((16,), 0, jnp.int32) + fill

        def pad(io, c2):
            pos = lanes16 + jnp.int32(io * 16)
            m = pos >= fillv
            sv = srcC[pl.ds(io * 16, 16)]
            lv = locC[pl.ds(io * 16, 16)]
            ev = eaC[pl.ds(io * 16, 16)]
            srcC[pl.ds(io * 16, 16)] = jnp.where(m, jnp.zeros((16,), jnp.int32), sv)
            locC[pl.ds(io * 16, 16)] = jnp.where(
                m, jnp.full((16,), SENTQ, jnp.int32), lv)
            eaC[pl.ds(io * 16, 16)] = jnp.where(m, jnp.zeros((16,), jnp.float32), ev)
            return c2
        lax.fori_loop(0, K // 16, pad, 0)
        drain(primed)
        wait_scatters()

        plsc.subcore_barrier()

        @pl.when(sid < NSUB - 1)
        def _():
            pltpu.sync_copy(S_num.at[pl.ds(sid * 784, 784)],
                            num_h.at[pl.ds(qb + sid * 784, 784)])
            pltpu.sync_copy(S_den.at[pl.ds(sid * 3136, 3136)],
                            den_h.at[pl.ds(qb * H + sid * 3136, 3136)])

        @pl.when(sid == NSUB - 1)
        def _():
            pltpu.sync_copy(S_num.at[pl.ds(15 * 784, 740)],
                            num_h.at[pl.ds(qb + 15 * 784, 740)])
            pltpu.sync_copy(S_den.at[pl.ds(15 * 3136, 2960)],
                            den_h.at[pl.ds(qb * H + 15 * 3136, 2960)])

        plsc.subcore_barrier()


_edge_kernel = functools.partial(
    pl.kernel,
    _edge_body,
    out_type=(jax.ShapeDtypeStruct((N, HID), jnp.float32),
              jax.ShapeDtypeStruct((N * H,), jnp.float32)),
    mesh=plsc.VectorSubcoreMesh(core_axis_name="c", subcore_axis_name="s"),
    compiler_params=pltpu.CompilerParams(needs_layout_passes=False,
                                         use_tc_tiling_on_sc=False),
    scratch_types=(
        pltpu.VMEM_SHARED((N * H,), jnp.float32),
        pltpu.VMEM_SHARED((R2 * H,), jnp.float32),
        pltpu.VMEM_SHARED((R2, HID), jnp.float32),
        pltpu.VMEM_SHARED((R2 * H,), jnp.float32),
        pltpu.VMEM((3, K), jnp.int32),
        pltpu.VMEM((3, K), jnp.int32),
        pltpu.VMEM((KR,), jnp.int32),
        pltpu.VMEM((KR,), jnp.int32),
        pltpu.VMEM((KR,), jnp.float32),
        pltpu.VMEM((K,), jnp.int32),
        pltpu.VMEM((K * H,), jnp.int32),
        pltpu.VMEM((K,), jnp.int32),
        pltpu.VMEM((K * H,), jnp.int32),
        pltpu.VMEM((K * H,), jnp.float32),
        pltpu.VMEM((K * H,), jnp.float32),
        pltpu.VMEM((K * H,), jnp.float32),
        pltpu.VMEM((K, HID), jnp.float32),
        pltpu.VMEM((16,), jnp.float32),
        pltpu.VMEM((16,), jnp.float32),
        pltpu.SemaphoreType.DMA,
        pltpu.SemaphoreType.DMA,
        pltpu.SemaphoreType.DMA,
        pltpu.SemaphoreType.DMA,
        pltpu.SemaphoreType.DMA,
        pltpu.SemaphoreType.DMA,
        pltpu.SemaphoreType.DMA,
    ),
)


def kernel(x, edge_index, edge_attr, batch, W_ne, b_ne, W_ee, b_ee, lin_w0, att_src0, att_dst0, att_edge0, lin_edge_w0, conv_b0, bn_g0, bn_b0, lin_w1, att_src1, att_dst1, att_edge1, lin_edge_w1, conv_b1, bn_g1, bn_b1, W_c1, b_c1, W_c2, b_c2):
    f32 = jnp.float32
    eaf = edge_attr.reshape(E)
    sde = jnp.concatenate(
        [edge_index,
         lax.bitcast_convert_type(eaf, jnp.int32).reshape(1, E)], axis=0)

    def prep(lin_w, a_s, a_d, a_e, lew):
        W3 = lin_w.reshape(H, C, HID)
        As = jnp.einsum("hck,hc->kh", W3, a_s)
        Ad = jnp.einsum("hck,hc->kh", W3, a_d)
        B3 = lew.reshape(H, C, HID)
        Bm = jnp.einsum("hck,hc->kh", B3, a_e)
        u = W_ee[:, 0] @ Bm
        v = b_ee @ Bm
        return As, Ad, jnp.tile(u, H), jnp.tile(v, H)

    bs = 1.0 / jnp.sqrt(1.0 + 1e-5)
    As0, Ad0, ut0, vt0 = prep(lin_w0, att_src0, att_dst0, att_edge0, lin_edge_w0)
    As1, Ad1, ut1, vt1 = prep(lin_w1, att_src1, att_dst1, att_edge1, lin_edge_w1)
    sc0 = (bs * bn_g0).reshape(1, HID)
    bi0 = (conv_b0 * bs * bn_g0 + bn_b0).reshape(1, HID)
    sc1 = (bs * bn_g1).reshape(1, HID)
    bi1 = (conv_b1 * bs * bn_g1 + bn_b1).reshape(1, HID)

    zn = jnp.zeros((784, HID), f32)
    zd = jnp.zeros((784 * H,), f32)
    ninf = jnp.full(((R2 - NQ) * H,), -jnp.inf, f32)

    h0, xl0, asrc0, adst0 = pl.pallas_call(
        _enc_dense_body,
        grid=(_NB,),
        in_specs=[
            pl.BlockSpec((_BN, IN), lambda i: (i, 0)),
            pl.BlockSpec((IN, HID), lambda i: (0, 0)),
            pl.BlockSpec((1, HID), lambda i: (0, 0)),
            pl.BlockSpec((HID, HID), lambda i: (0, 0)),
            pl.BlockSpec((HID, H), lambda i: (0, 0)),
            pl.BlockSpec((HID, H), lambda i: (0, 0)),
        ],
        out_specs=[
            pl.BlockSpec((_BN, HID), lambda i: (i, 0)),
            pl.BlockSpec((_BN, HID), lambda i: (i, 0)),
            pl.BlockSpec((_BN, H), lambda i: (i, 0)),
            pl.BlockSpec((_BN, H), lambda i: (i, 0)),
        ],
        out_shape=[
            jax.ShapeDtypeStruct((N, HID), f32),
            jax.ShapeDtypeStruct((N, HID), f32),
            jax.ShapeDtypeStruct((N, H), f32),
            jax.ShapeDtypeStruct((N, H), f32),
        ],
    )(x, W_ne.T, b_ne.reshape(1, HID), lin_w0.T, As0, Ad0)

    num0, den0 = _edge_kernel()(sde,
                                asrc0.reshape(N * H), adst0.reshape(N * H),
                                xl0, ut0, vt0, zn, zd, ninf)
    den0 = den0.reshape(N, H)

    h1, xl1, asrc1, adst1 = pl.pallas_call(
        _epi_dense_body,
        grid=(_NB,),
        in_specs=[
            pl.BlockSpec((_BN, HID), lambda i: (i, 0)),
            pl.BlockSpec((_BN, H), lambda i: (i, 0)),
            pl.BlockSpec((_BN, HID), lambda i: (i, 0)),
            pl.BlockSpec((1, HID), lambda i: (0, 0)),
            pl.BlockSpec((1, HID), lambda i: (0, 0)),
            pl.BlockSpec((HID, HID), lambda i: (0, 0)),
            pl.BlockSpec((HID, H), lambda i: (0, 0)),
            pl.BlockSpec((HID, H), lambda i: (0, 0)),
        ],
        out_specs=[
            pl.BlockSpec((_BN, HID), lambda i: (i, 0)),
            pl.BlockSpec((_BN, HID), lambda i: (i, 0)),
            pl.BlockSpec((_BN, H), lambda i: (i, 0)),
            pl.BlockSpec((_BN, H), lambda i: (i, 0)),
        ],
        out_shape=[
            jax.ShapeDtypeStruct((N, HID), f32),
            jax.ShapeDtypeStruct((N, HID), f32),
            jax.ShapeDtypeStruct((N, H), f32),
            jax.ShapeDtypeStruct((N, H), f32),
        ],
    )(num0, den0, h0, sc0, bi0, lin_w1.T, As1, Ad1)

    num1, den1 = _edge_kernel()(sde,
                                asrc1.reshape(N * H), adst1.reshape(N * H),
                                xl1, ut1, vt1, zn, zd, ninf)
    den1 = den1.reshape(N, H)

    out, _sums, _cnt = pl.pallas_call(
        _final_body,
        grid=(_NB,),
        in_specs=[
            pl.BlockSpec((_BN, HID), lambda i: (i, 0)),
            pl.BlockSpec((_BN, H), lambda i: (i, 0)),
            pl.BlockSpec((_BN, HID), lambda i: (i, 0)),
            pl.BlockSpec((1, HID), lambda i: (0, 0)),
            pl.BlockSpec((1, HID), lambda i: (0, 0)),
            pl.BlockSpec((1, 1, _BN), lambda i: (i, 0, 0)),
            pl.BlockSpec((HID, HID // 2), lambda i: (0, 0)),
            pl.BlockSpec((1, HID // 2), lambda i: (0, 0)),
            pl.BlockSpec((HID // 2, 1), lambda i: (0, 0)),
            pl.BlockSpec((1, 1), lambda i: (0, 0)),
        ],
        out_specs=[
            pl.BlockSpec((1, G), lambda i: (0, 0)),
            pl.BlockSpec((G, HID), lambda i: (0, 0)),
            pl.BlockSpec((G, HID), lambda i: (0, 0)),
        ],
        out_shape=[
            jax.ShapeDtypeStruct((1, G), f32),
            jax.ShapeDtypeStruct((G, HID), f32),
            jax.ShapeDtypeStruct((G, HID), f32),
        ],
    )(num1, den1, h1, sc1, bi1, batch.reshape(_NB, 1, _BN),
      W_c1.T, b_c1.reshape(1, HID // 2), W_c2.T, b_c2.reshape(1, 1))

    return out.reshape(G)


# speedup vs baseline: 93.1163x; 1.0123x over previous
import functools

import jax
import jax.numpy as jnp
from jax import lax
from jax.experimental import pallas as pl
from jax.experimental.pallas import tpu as pltpu
from jax.experimental.pallas import tpu_sc as plsc

N = 50000
E = 800000
IN = 128
HID = 64
H = 4
C = 16
G = 64

NSUB = 16
NCORE = 2
N2 = N // 2
NQ = N // 4
R2 = 12544
SENTQ = NQ
EPW = E // NSUB
K = 400
KP = 512
NCH = EPW // K

_NB = 125
_BN = 400


def _enc_dense_body(x_ref, wne_ref, bne_ref, lwT_ref, as_ref, ad_ref,
                    h_ref, xl_ref, asrc_ref, adst_ref):
    h = jnp.dot(x_ref[...], wne_ref[...], preferred_element_type=jnp.float32)
    h = h + bne_ref[...]
    h_ref[...] = h
    xl_ref[...] = jnp.dot(h, lwT_ref[...], preferred_element_type=jnp.float32)
    asrc_ref[...] = jnp.dot(h, as_ref[...], preferred_element_type=jnp.float32)
    adst_ref[...] = jnp.dot(h, ad_ref[...], preferred_element_type=jnp.float32)


def _epi_dense_body(num_ref, den_ref, hin_ref, sc_ref, bi_ref,
                    lwT_ref, as_ref, ad_ref,
                    h_ref, xl_ref, asrc_ref, adst_ref):
    heads = lax.broadcasted_iota(jnp.int32, (H, HID), 1) // C
    R = (heads == lax.broadcasted_iota(jnp.int32, (H, HID), 0)).astype(jnp.float32)
    denx = jnp.dot(den_ref[...], R, preferred_element_type=jnp.float32)
    gat = num_ref[...] / (denx + 1e-16)
    h2 = gat * sc_ref[...] + bi_ref[...]
    h = jnp.maximum(h2, 0.0) + hin_ref[...]
    h_ref[...] = h
    xl_ref[...] = jnp.dot(h, lwT_ref[...], preferred_element_type=jnp.float32)
    asrc_ref[...] = jnp.dot(h, as_ref[...], preferred_element_type=jnp.float32)
    adst_ref[...] = jnp.dot(h, ad_ref[...], preferred_element_type=jnp.float32)


def _final_body(num_ref, den_ref, hin_ref, sc_ref, bi_ref, batch_ref,
                wc1_ref, bc1_ref, wc2_ref, bc2_ref,
                out_ref, sums_ref, cnt_ref):
    i = pl.program_id(0)
    heads = lax.broadcasted_iota(jnp.int32, (H, HID), 1) // C
    R = (heads == lax.broadcasted_iota(jnp.int32, (H, HID), 0)).astype(jnp.float32)
    denx = jnp.dot(den_ref[...], R, preferred_element_type=jnp.float32)
    gat = num_ref[...] / (denx + 1e-16)
    h2 = gat * sc_ref[...] + bi_ref[...]
    h = jnp.maximum(h2, 0.0) + hin_ref[...]

    b = batch_ref[0, 0, :]
    oh = (b[:, None] == lax.broadcasted_iota(jnp.int32, (_BN, G), 1)).astype(jnp.float32)
    sums_d = lax.dot_general(oh, h, (((0,), (0,)), ((), ())),
                             preferred_element_type=jnp.float32)
    cnt_d = lax.dot_general(oh, jnp.ones((_BN, HID), jnp.float32),
                            (((0,), (0,)), ((), ())),
                            preferred_element_type=jnp.float32)

    @pl.when(i == 0)
    def _():
        sums_ref[...] = sums_d
        cnt_ref[...] = cnt_d

    @pl.when(i > 0)
    def _():
        sums_ref[...] += sums_d
        cnt_ref[...] += cnt_d

    @pl.when(i == _NB - 1)
    def _():
        pooled = sums_ref[...] / jnp.maximum(cnt_ref[...], 1.0)
        hc = jnp.dot(pooled, wc1_ref[...], preferred_element_type=jnp.float32)
        hc = jnp.maximum(hc + bc1_ref[...], 0.0)
        o = jnp.dot(hc, wc2_ref[...], preferred_element_type=jnp.float32)
        o = jax.nn.sigmoid(o + bc2_ref[...])
        out_ref[...] = o.T


KR = 2 * K + 16


def _edge_body(sde_h, asrc_h, adst_h, xl_h, ut_h, vt_h,
               zn_h, zd_h, ninf_h,
               num_h, den_h,
               S_asrc, S_adst, S_num, S_den,
               sdeA, sdeB, srcC, locC, eaC,
               idxs_v, idxa_v, idxd_v, idxe_v,
               ag_v, bg_v, ex_v, xlG, uv_v, vv_v,
               sem0, sem1, sem2, semLA, semLB, semS1, semS2):
    cid = lax.axis_index("c")
    sid = lax.axis_index("s")

    @pl.when(sid < NSUB - 1)
    def _():
        pltpu.sync_copy(asrc_h.at[pl.ds(sid * 12512, 12512)],
                        S_asrc.at[pl.ds(sid * 12512, 12512)])

    @pl.when(sid == NSUB - 1)
    def _():
        pltpu.sync_copy(asrc_h.at[pl.ds(15 * 12512, 12320)],
                        S_asrc.at[pl.ds(15 * 12512, 12320)])

    pltpu.sync_copy(ut_h, uv_v)
    pltpu.sync_copy(vt_h, vv_v)

    utile = uv_v[...]
    vtile = vv_v[...]
    expand4 = lax.iota(jnp.int32, 16) // jnp.int32(4)
    headpat = lax.iota(jnp.int32, 16) % jnp.int32(4)
    lanes16 = lax.iota(jnp.int32, 16)

    def wait_scatters():
        pltpu.make_async_copy(ex_v, S_den.at[idxe_v], semS1).wait()
        pltpu.make_async_copy(xlG, S_num.at[idxd_v], semS2).wait()

    def drain(primed):
        lax.cond(primed != 0, wait_scatters, lambda: None)

        def build(io, c2):
            sv = srcC[pl.ds(io * 16, 16)]
            lv = locC[pl.ds(io * 16, 16)]
            idxs_v[pl.ds(io * 16, 16)] = sv
            idxd_v[pl.ds(io * 16, 16)] = lv
            for q in range(4):
                idxc = expand4 + jnp.int32(4 * q)
                sexp = jnp.take_along_axis(sv, idxc, axis=0,
                                           mode="promise_in_bounds")
                lexp = jnp.take_along_axis(lv, idxc, axis=0,
                                           mode="promise_in_bounds")
                j0 = io * 64 + q * 16
                idxa_v[pl.ds(j0, 16)] = sexp * H + headpat
                idxe_v[pl.ds(j0, 16)] = lexp * H + headpat
            return c2
        lax.fori_loop(0, K // 16, build, 0)

        cp0 = pltpu.async_copy(S_asrc.at[idxa_v], ag_v, sem0)
        cp1 = pltpu.async_copy(S_adst.at[idxe_v], bg_v, sem1)
        cp2 = pltpu.async_copy(xl_h.at[idxs_v], xlG, sem2)
        cp0.wait()
        cp1.wait()
        cp2.wait()

        def exloop(io, c2):
            eav = eaC[pl.ds(io * 16, 16)]
            for q in range(4):
                idxc = expand4 + jnp.int32(4 * q)
                aeq = jnp.take_along_axis(eav, idxc, axis=0,
                                          mode="promise_in_bounds")
                aeq = aeq * utile + vtile
                j0 = io * 64 + q * 16
                a = ag_v[pl.ds(j0, 16)] + bg_v[pl.ds(j0, 16)] + aeq
                a = jnp.maximum(a, a * 0.2)
                ex_v[pl.ds(j0, 16)] = jnp.exp(a)
            return c2
        lax.fori_loop(0, K // 16, exloop, 0, unroll=2)

        def mulloop(io, c2):
            for q in range(4):
                exq = ex_v[pl.ds(io * 64 + q * 16, 16)]
                for t in range(4):
                    e = io * 16 + q * 4 + t
                    for hh in range(H):
                        splat = jnp.take_along_axis(
                            exq, jnp.full((16,), t * 4 + hh, jnp.int32),
                            axis=0, mode="promise_in_bounds")
                        v = xlG[e, pl.ds(hh * 16, 16)]
                        xlG[e, pl.ds(hh * 16, 16)] = v * splat
            return c2
        lax.fori_loop(0, K // 16, mulloop, 0, unroll=2)

        pltpu.async_copy(ex_v, S_den.at[idxe_v], semS1, add=True)
        pltpu.async_copy(xlG, S_num.at[idxd_v], semS2, add=True)

    for p in range(2):
        qb = cid * N2 + p * NQ

        @pl.when(sid < NSUB - 1)
        def _():
            pltpu.sync_copy(adst_h.at[pl.ds(qb * H + sid * 3136, 3136)],
                            S_adst.at[pl.ds(sid * 3136, 3136)])

        @pl.when(sid == NSUB - 1)
        def _():
            pltpu.sync_copy(adst_h.at[pl.ds(qb * H + 15 * 3136, 2960)],
                            S_adst.at[pl.ds(15 * 3136, 2960)])

        @pl.when(sid == 0)
        def _():
            pltpu.sync_copy(ninf_h, S_adst.at[pl.ds(NQ * H, (R2 - NQ) * H)])

        pltpu.sync_copy(zn_h, S_num.at[pl.ds(sid * 784, 784)])
        pltpu.sync_copy(zd_h, S_den.at[pl.ds(sid * 3136, 3136)])

        plsc.subcore_barrier()

        qbv = jnp.full((16,), qb, jnp.int32)

        def fire_load(j, buf, sem):
            base = pl.multiple_of(sid * EPW + j * K, 8)
            return pltpu.async_copy(sde_h.at[:, pl.ds(base, K)], buf, sem)

        def wait_load(j, buf, sem):
            base = pl.multiple_of(sid * EPW + j * K, 8)
            pltpu.make_async_copy(sde_h.at[:, pl.ds(base, K)], buf, sem).wait()

        def route_chunk(buf, carry):
            fill, primed = carry

            def route(io, f):
                sv = buf[0, pl.ds(io * 16, 16)]
                d = buf[1, pl.ds(io * 16, 16)]
                eav = plsc.bitcast(buf[2, pl.ds(io * 16, 16)], jnp.float32)
                own = (d >= qbv) & (d < qbv + jnp.int32(NQ))
                loc = d - qbv
                plsc.store_compressed(srcC.at[pl.ds(f, 16)], sv, mask=own)
                plsc.store_compressed(locC.at[pl.ds(f, 16)], loc, mask=own)
                plsc.store_compressed(eaC.at[pl.ds(f, 16)], eav, mask=own)
                cnt = plsc.all_reduce_population_count(own)
                return f + (cnt[0] if getattr(cnt, "ndim", 0) else cnt)
            fill = lax.fori_loop(0, K // 16, route, fill)

            def do_drain(ff, pr):
                drain(pr)

                def shift(io, c2):
                    srcC[pl.ds(io * 16, 16)] = srcC[pl.ds(K + io * 16, 16)]
                    locC[pl.ds(io * 16, 16)] = locC[pl.ds(K + io * 16, 16)]
                    eaC[pl.ds(io * 16, 16)] = eaC[pl.ds(K + io * 16, 16)]
                    return c2
                lax.fori_loop(0, K // 16, shift, 0)
                return ff - jnp.int32(K), jnp.int32(1)

            return lax.cond(fill >= K, do_drain,
                            lambda ff, pr: (ff, pr), fill, primed)

        fire_load(0, sdeA, semLA)

        def chunk2(j2, carry):
            wait_load(2 * j2, sdeA, semLA)
            fire_load(2 * j2 + 1, sdeB, semLB)
            carry = route_chunk(sdeA, carry)
            wait_load(2 * j2 + 1, sdeB, semLB)
            fire_load(2 * j2 + 2, sdeA, semLA)
            return route_chunk(sdeB, carry)

        fill, primed = lax.fori_loop(0, NCH // 2, chunk2,
                                     (jnp.int32(0), jnp.int32(0)))
        wait_load(NCH - 1, sdeA, semLA)
        fill, primed = route_chunk(sdeA, (fill, primed))

        fillv = jnp.full((16,), 0, jnp.int32) + fill

        def pad(io, c2):
            pos = lanes16 + jnp.int32(io * 16)
            m = pos >= fillv
            sv = srcC[pl.ds(io * 16, 16)]
            lv = locC[pl.ds(io * 16, 16)]
            ev = eaC[pl.ds(io * 16, 16)]
            srcC[pl.ds(io * 16, 16)] = jnp.where(m, jnp.zeros((16,), jnp.int32), sv)
            locC[pl.ds(io * 16, 16)] = jnp.where(
                m, jnp.full((16,), SENTQ, jnp.int32), lv)
            eaC[pl.ds(io * 16, 16)] = jnp.where(m, jnp.zeros((16,), jnp.float32), ev)
            return c2
        lax.fori_loop(0, K // 16, pad, 0)
        drain(primed)
        wait_scatters()

        plsc.subcore_barrier()

        @pl.when(sid < NSUB - 1)
        def _():
            pltpu.sync_copy(S_num.at[pl.ds(sid * 784, 784)],
                            num_h.at[pl.ds(qb + sid * 784, 784)])
            pltpu.sync_copy(S_den.at[pl.ds(sid * 3136, 3136)],
                            den_h.at[pl.ds(qb * H + sid * 3136, 3136)])

        @pl.when(sid == NSUB - 1)
        def _():
            pltpu.sync_copy(S_num.at[pl.ds(15 * 784, 740)],
                            num_h.at[pl.ds(qb + 15 * 784, 740)])
            pltpu.sync_copy(S_den.at[pl.ds(15 * 3136, 2960)],
                            den_h.at[pl.ds(qb * H + 15 * 3136, 2960)])

        plsc.subcore_barrier()


_edge_kernel = functools.partial(
    pl.kernel,
    _edge_body,
    out_type=(jax.ShapeDtypeStruct((N, HID), jnp.float32),
              jax.ShapeDtypeStruct((N * H,), jnp.float32)),
    mesh=plsc.VectorSubcoreMesh(core_axis_name="c", subcore_axis_name="s"),
    compiler_params=pltpu.CompilerParams(needs_layout_passes=False,
                                         use_tc_tiling_on_sc=False),
    scratch_types=(
        pltpu.VMEM_SHARED((N * H,), jnp.float32),
        pltpu.VMEM_SHARED((R2 * H,), jnp.float32),
        pltpu.VMEM_SHARED((R2, HID), jnp.float32),
        pltpu.VMEM_SHARED((R2 * H,), jnp.float32),
        pltpu.VMEM((3, K), jnp.int32),
        pltpu.VMEM((3, K), jnp.int32),
        pltpu.VMEM((KR,), jnp.int32),
        pltpu.VMEM((KR,), jnp.int32),
        pltpu.VMEM((KR,), jnp.float32),
        pltpu.VMEM((K,), jnp.int32),
        pltpu.VMEM((K * H,), jnp.int32),
        pltpu.VMEM((K,), jnp.int32),
        pltpu.VMEM((K * H,), jnp.int32),
        pltpu.VMEM((K * H,), jnp.float32),
        pltpu.VMEM((K * H,), jnp.float32),
        pltpu.VMEM((K * H,), jnp.float32),
        pltpu.VMEM((K, HID), jnp.float32),
        pltpu.VMEM((16,), jnp.float32),
        pltpu.VMEM((16,), jnp.float32),
        pltpu.SemaphoreType.DMA,
        pltpu.SemaphoreType.DMA,
        pltpu.SemaphoreType.DMA,
        pltpu.SemaphoreType.DMA,
        pltpu.SemaphoreType.DMA,
        pltpu.SemaphoreType.DMA,
        pltpu.SemaphoreType.DMA,
    ),
)


def kernel(x, edge_index, edge_attr, batch, W_ne, b_ne, W_ee, b_ee, lin_w0, att_src0, att_dst0, att_edge0, lin_edge_w0, conv_b0, bn_g0, bn_b0, lin_w1, att_src1, att_dst1, att_edge1, lin_edge_w1, conv_b1, bn_g1, bn_b1, W_c1, b_c1, W_c2, b_c2):
    f32 = jnp.float32
    eaf = edge_attr.reshape(E)
    sde = jnp.concatenate(
        [edge_index,
         lax.bitcast_convert_type(eaf, jnp.int32).reshape(1, E)], axis=0)

    def prep(lin_w, a_s, a_d, a_e, lew):
        W3 = lin_w.reshape(H, C, HID)
        As = jnp.einsum("hck,hc->kh", W3, a_s)
        Ad = jnp.einsum("hck,hc->kh", W3, a_d)
        B3 = lew.reshape(H, C, HID)
        Bm = jnp.einsum("hck,hc->kh", B3, a_e)
        u = W_ee[:, 0] @ Bm
        v = b_ee @ Bm
        return As, Ad, jnp.tile(u, H), jnp.tile(v, H)

    bs = 1.0 / jnp.sqrt(1.0 + 1e-5)
    As0, Ad0, ut0, vt0 = prep(lin_w0, att_src0, att_dst0, att_edge0, lin_edge_w0)
    As1, Ad1, ut1, vt1 = prep(lin_w1, att_src1, att_dst1, att_edge1, lin_edge_w1)
    sc0 = (bs * bn_g0).reshape(1, HID)
    bi0 = (conv_b0 * bs * bn_g0 + bn_b0).reshape(1, HID)
    sc1 = (bs * bn_g1).reshape(1, HID)
    bi1 = (conv_b1 * bs * bn_g1 + bn_b1).reshape(1, HID)

    zn = jnp.zeros((784, HID), f32)
    zd = jnp.zeros((784 * H,), f32)
    ninf = jnp.full(((R2 - NQ) * H,), -jnp.inf, f32)

    h0, xl0, asrc0, adst0 = pl.pallas_call(
        _enc_dense_body,
        grid=(_NB,),
        in_specs=[
            pl.BlockSpec((_BN, IN), lambda i: (i, 0)),
            pl.BlockSpec((IN, HID), lambda i: (0, 0)),
            pl.BlockSpec((1, HID), lambda i: (0, 0)),
            pl.BlockSpec((HID, HID), lambda i: (0, 0)),
            pl.BlockSpec((HID, H), lambda i: (0, 0)),
            pl.BlockSpec((HID, H), lambda i: (0, 0)),
        ],
        out_specs=[
            pl.BlockSpec((_BN, HID), lambda i: (i, 0)),
            pl.BlockSpec((_BN, HID), lambda i: (i, 0)),
            pl.BlockSpec((_BN, H), lambda i: (i, 0)),
            pl.BlockSpec((_BN, H), lambda i: (i, 0)),
        ],
        out_shape=[
            jax.ShapeDtypeStruct((N, HID), f32),
            jax.ShapeDtypeStruct((N, HID), f32),
            jax.ShapeDtypeStruct((N, H), f32),
            jax.ShapeDtypeStruct((N, H), f32),
        ],
    )(x, W_ne.T, b_ne.reshape(1, HID), lin_w0.T, As0, Ad0)

    num0, den0 = _edge_kernel()(sde,
                                asrc0.reshape(N * H), adst0.reshape(N * H),
                                xl0, ut0, vt0, zn, zd, ninf)
    den0 = den0.reshape(N, H)

    h1, xl1, asrc1, adst1 = pl.pallas_call(
        _epi_dense_body,
        grid=(_NB,),
        in_specs=[
            pl.BlockSpec((_BN, HID), lambda i: (i, 0)),
            pl.BlockSpec((_BN, H), lambda i: (i, 0)),
            pl.BlockSpec((_BN, HID), lambda i: (i, 0)),
            pl.BlockSpec((1, HID), lambda i: (0, 0)),
            pl.BlockSpec((1, HID), lambda i: (0, 0)),
            pl.BlockSpec((HID, HID), lambda i: (0, 0)),
            pl.BlockSpec((HID, H), lambda i: (0, 0)),
            pl.BlockSpec((HID, H), lambda i: (0, 0)),
        ],
        out_specs=[
            pl.BlockSpec((_BN, HID), lambda i: (i, 0)),
            pl.BlockSpec((_BN, HID), lambda i: (i, 0)),
            pl.BlockSpec((_BN, H), lambda i: (i, 0)),
            pl.BlockSpec((_BN, H), lambda i: (i, 0)),
        ],
        out_shape=[
            jax.ShapeDtypeStruct((N, HID), f32),
            jax.ShapeDtypeStruct((N, HID), f32),
            jax.ShapeDtypeStruct((N, H), f32),
            jax.ShapeDtypeStruct((N, H), f32),
        ],
    )(num0, den0, h0, sc0, bi0, lin_w1.T, As1, Ad1)

    num1, den1 = _edge_kernel()(sde,
                                asrc1.reshape(N * H), adst1.reshape(N * H),
                                xl1, ut1, vt1, zn, zd, ninf)
    den1 = den1.reshape(N, H)

    out, _sums, _cnt = pl.pallas_call(
        _final_body,
        grid=(_NB,),
        in_specs=[
            pl.BlockSpec((_BN, HID), lambda i: (i, 0)),
            pl.BlockSpec((_BN, H), lambda i: (i, 0)),
            pl.BlockSpec((_BN, HID), lambda i: (i, 0)),
            pl.BlockSpec((1, HID), lambda i: (0, 0)),
            pl.BlockSpec((1, HID), lambda i: (0, 0)),
            pl.BlockSpec((1, 1, _BN), lambda i: (i, 0, 0)),
            pl.BlockSpec((HID, HID // 2), lambda i: (0, 0)),
            pl.BlockSpec((1, HID // 2), lambda i: (0, 0)),
            pl.BlockSpec((HID // 2, 1), lambda i: (0, 0)),
            pl.BlockSpec((1, 1), lambda i: (0, 0)),
        ],
        out_specs=[
            pl.BlockSpec((1, G), lambda i: (0, 0)),
            pl.BlockSpec((G, HID), lambda i: (0, 0)),
            pl.BlockSpec((G, HID), lambda i: (0, 0)),
        ],
        out_shape=[
            jax.ShapeDtypeStruct((1, G), f32),
            jax.ShapeDtypeStruct((G, HID), f32),
            jax.ShapeDtypeStruct((G, HID), f32),
        ],
    )(num1, den1, h1, sc1, bi1, batch.reshape(_NB, 1, _BN),
      W_c1.T, b_c1.reshape(1, HID // 2), W_c2.T, b_c2.reshape(1, 1))

    return out.reshape(G)
